# Initial kernel scaffold; baseline (speedup 1.0000x reference)
import functools

import jax
import jax.numpy as jnp
from jax import lax
from jax.experimental import pallas as pl
from jax.experimental.pallas import tpu as pltpu
from jax.experimental.pallas import tpu_sc as plsc

N = 50000
E = 800000
R = 8
D = 64
B = 4096

NC = 2
NS = 16

HALF = 25088
NPAD = 2 * HALF
HROWS = 25600
DUMMY = HALF
EPAD = 819200
EPC = EPAD // NS
CH = 2048
NCHUNK = EPC // CH
BN = 1000
BQ = 512

_HIGH = jax.lax.Precision.HIGHEST


def _mesh():
    return plsc.VectorSubcoreMesh(core_axis_name="c", subcore_axis_name="s",
                                  num_cores=NC, num_subcores=NS)


def _deg_sc(dst_p):
    @functools.partial(
        pl.kernel,
        out_type=jax.ShapeDtypeStruct((NPAD, 16), jnp.float32),
        mesh=_mesh(),
        scratch_types=[
            pltpu.VMEM_SHARED((HROWS, 16), jnp.float32),
            pltpu.VMEM((CH,), jnp.int32),
            pltpu.VMEM((16, 128), jnp.int32),
            pltpu.VMEM((128, 16), jnp.float32),
            pltpu.VMEM((200, 16), jnp.float32),
        ],
    )
    def k(dst_hbm, deg_hbm, sdeg, dstb, locb, onesb, zb):
        cid = lax.axis_index("c")
        sid = lax.axis_index("s")
        base = cid * HALF
        ii = lax.iota(jnp.int32, 16)
        onerow = jnp.where(ii == 0, 1.0, 0.0).astype(jnp.float32)
        zrow = jnp.zeros((16,), jnp.float32)

        @pl.loop(0, 200)
        def _(i):
            zb[i, pl.ds(0, 16)] = zrow

        @pl.loop(0, 128)
        def _(i):
            onesb[i, pl.ds(0, 16)] = onerow

        @pl.loop(0, 8)
        def _(t):
            pltpu.sync_copy(zb, sdeg.at[pl.ds(sid * 1600 + t * 200, 200)])

        plsc.subcore_barrier()

        ebase = sid * EPC

        @pl.loop(0, NCHUNK)
        def _(t):
            pltpu.sync_copy(dst_hbm.at[pl.ds(ebase + t * CH, CH)], dstb)

            @pl.loop(0, 16)
            def _(j):
                @pl.loop(0, 8)
                def _(v):
                    dd = dstb[pl.ds(j * 128 + v * 16, 16)]
                    lo = dd - base
                    ok = (lo >= 0) & (lo < HALF)
                    locb[j, pl.ds(v * 16, 16)] = jnp.where(ok, lo, DUMMY)

            @pl.loop(0, 16)
            def _(j):
                pltpu.sync_copy(onesb, sdeg.at[locb.at[j]], add=True)

        plsc.subcore_barrier()
        pltpu.sync_copy(sdeg.at[pl.ds(sid * 1568, 1568)],
                        deg_hbm.at[pl.ds(base + sid * 1568, 1568)])

    return k(dst_p)


def _agg_sc(ytab, src_p, dst_p, typ_p):
    @functools.partial(
        pl.kernel,
        out_type=jax.ShapeDtypeStruct((NPAD, D), jnp.float32),
        mesh=_mesh(),
        scratch_types=[
            pltpu.VMEM_SHARED((HROWS, D), jnp.float32),
            pltpu.VMEM((CH,), jnp.int32),
            pltpu.VMEM((CH,), jnp.int32),
            pltpu.VMEM((CH,), jnp.int32),
            pltpu.VMEM((16, 128), jnp.int32),
            pltpu.VMEM((16, 128), jnp.int32),
            pltpu.VMEM((128, D), jnp.float32),
            pltpu.VMEM((200, D), jnp.float32),
            pltpu.SemaphoreType.DMA,
        ],
    )
    def k(ytab_hbm, src_hbm, dst_hbm, typ_hbm, agg_hbm,
          sagg, srcb, dstb, typb, gidx, locb, rowb, zb, sem):
        cid = lax.axis_index("c")
        sid = lax.axis_index("s")
        base = cid * HALF
        zrow = jnp.zeros((16,), jnp.float32)

        @pl.loop(0, 200)
        def _(i):
            @pl.loop(0, D // 16)
            def _(j):
                zb[i, pl.ds(j * 16, 16)] = zrow

        @pl.loop(0, 8)
        def _(t):
            pltpu.sync_copy(zb, sagg.at[pl.ds(sid * 1600 + t * 200, 200)])

        plsc.subcore_barrier()

        ebase = sid * EPC

        @pl.loop(0, NCHUNK)
        def _(t):
            off = ebase + t * CH
            pltpu.sync_copy(src_hbm.at[pl.ds(off, CH)], srcb)
            pltpu.sync_copy(typ_hbm.at[pl.ds(off, CH)], typb)
            pltpu.sync_copy(dst_hbm.at[pl.ds(off, CH)], dstb)

            @pl.loop(0, 16)
            def _(j):
                @pl.loop(0, 8)
                def _(v):
                    s = srcb[pl.ds(j * 128 + v * 16, 16)]
                    ty = typb[pl.ds(j * 128 + v * 16, 16)]
                    dd = dstb[pl.ds(j * 128 + v * 16, 16)]
                    gidx[j, pl.ds(v * 16, 16)] = ty * N + s
                    lo = dd - base
                    ok = (lo >= 0) & (lo < HALF)
                    locb[j, pl.ds(v * 16, 16)] = jnp.where(ok, lo, DUMMY)

            @pl.loop(0, 16)
            def _(j):
                pltpu.async_copy(ytab_hbm.at[gidx.at[j]], rowb, sem).wait()
                pltpu.sync_copy(rowb, sagg.at[locb.at[j]], add=True)

        plsc.subcore_barrier()
        pltpu.sync_copy(sagg.at[pl.ds(sid * 1568, 1568)],
                        agg_hbm.at[pl.ds(base + sid * 1568, 1568)])

    return k(ytab, src_p, dst_p, typ_p)


def _zgather_sc(x, heads, tails):
    rows = B // (NC * NS)

    @functools.partial(
        pl.kernel,
        out_type=(jax.ShapeDtypeStruct((B, D), jnp.float32),
                  jax.ShapeDtypeStruct((B, D), jnp.float32)),
        mesh=_mesh(),
        scratch_types=[
            pltpu.VMEM((rows,), jnp.int32),
            pltpu.VMEM((rows, D), jnp.float32),
            pltpu.SemaphoreType.DMA,
        ],
    )
    def k(x_hbm, h_hbm, t_hbm, zh_hbm, zt_hbm, idxb, rowb, sem):
        cid = lax.axis_index("c")
        sid = lax.axis_index("s")
        wid = sid * NC + cid
        off = wid * rows
        pltpu.sync_copy(h_hbm.at[pl.ds(off, rows)], idxb)
        pltpu.async_copy(x_hbm.at[idxb], rowb, sem).wait()
        pltpu.sync_copy(rowb, zh_hbm.at[pl.ds(off, rows)])
        pltpu.sync_copy(t_hbm.at[pl.ds(off, rows)], idxb)
        pltpu.async_copy(x_hbm.at[idxb], rowb, sem).wait()
        pltpu.sync_copy(rowb, zt_hbm.at[pl.ds(off, rows)])

    return k(x, heads, tails)


def _proj_body(nf_ref, w_ref, b_ref, o_ref):
    o_ref[...] = jnp.maximum(
        jnp.dot(nf_ref[...], w_ref[...], precision=_HIGH,
                preferred_element_type=jnp.float32) + b_ref[...], 0.0)


def _proj_tc(node_feat, W_in, b_in):
    return pl.pallas_call(
        _proj_body,
        grid=(N // BN,),
        in_specs=[
            pl.BlockSpec((BN, 4), lambda i: (i, 0)),
            pl.BlockSpec((4, D), lambda i: (0, 0)),
            pl.BlockSpec((1, D), lambda i: (0, 0)),
        ],
        out_specs=pl.BlockSpec((BN, D), lambda i: (i, 0)),
        out_shape=jax.ShapeDtypeStruct((N, D), jnp.float32),
    )(node_feat, W_in, b_in)


def _ytab_body(x_ref, rw_ref, sw_ref, sb_ref, y_ref, self_ref):
    xb = x_ref[...]
    y_ref[0] = jnp.dot(xb, rw_ref[0], precision=_HIGH,
                       preferred_element_type=jnp.float32)

    @pl.when(pl.program_id(1) == 0)
    def _():
        self_ref[...] = jnp.dot(xb, sw_ref[...], precision=_HIGH,
                                preferred_element_type=jnp.float32) + sb_ref[...]


def _ytab_tc(x, rel_w, sl_w, sl_b):
    return pl.pallas_call(
        _ytab_body,
        grid=(N // BN, R),
        in_specs=[
            pl.BlockSpec((BN, D), lambda i, r: (i, 0)),
            pl.BlockSpec((1, D, D), lambda i, r: (r, 0, 0)),
            pl.BlockSpec((D, D), lambda i, r: (0, 0)),
            pl.BlockSpec((1, D), lambda i, r: (0, 0)),
        ],
        out_specs=[
            pl.BlockSpec((1, BN, D), lambda i, r: (r, i, 0)),
            pl.BlockSpec((BN, D), lambda i, r: (i, 0)),
        ],
        out_shape=[
            jax.ShapeDtypeStruct((R, N, D), jnp.float32),
            jax.ShapeDtypeStruct((N, D), jnp.float32),
        ],
    )(x, rel_w, sl_w, sl_b)


def _comb_body(self_ref, agg_ref, deg_ref, o_ref):
    deg = deg_ref[:, 0:1]
    inv = 1.0 / jnp.maximum(deg, 1.0)
    o_ref[...] = jnp.maximum(self_ref[...] + agg_ref[...] * inv, 0.0)


def _comb_tc(self_x, agg, degp):
    return pl.pallas_call(
        _comb_body,
        grid=(N // BN,),
        in_specs=[
            pl.BlockSpec((BN, D), lambda i: (i, 0)),
            pl.BlockSpec((BN, D), lambda i: (i, 0)),
            pl.BlockSpec((BN, 16), lambda i: (i, 0)),
        ],
        out_specs=pl.BlockSpec((BN, D), lambda i: (i, 0)),
        out_shape=jax.ShapeDtypeStruct((N, D), jnp.float32),
    )(self_x, agg, degp)


def _scorer_body(zh_ref, zt_ref, rels_ref, re_ref, w1_ref, b1_ref, w2_ref,
                 b2_ref, o_ref):
    r = rels_ref[0, 0, :]
    oh = (lax.broadcasted_iota(jnp.int32, (BQ, R), 1) == r[:, None])
    re = jnp.dot(oh.astype(jnp.float32), re_ref[...], precision=_HIGH,
                 preferred_element_type=jnp.float32)
    w1 = w1_ref[...]
    h = (jnp.dot(zh_ref[...], w1[0:D], precision=_HIGH,
                 preferred_element_type=jnp.float32)
         + jnp.dot(zt_ref[...], w1[D:2 * D], precision=_HIGH,
                   preferred_element_type=jnp.float32)
         + jnp.dot(re, w1[2 * D:3 * D], precision=_HIGH,
                   preferred_element_type=jnp.float32)
         + b1_ref[...])
    h = jnp.maximum(h, 0.0)
    s = jnp.sum(h * w2_ref[...], axis=1, keepdims=True) + b2_ref[...]
    o_ref[...] = s


def _scorer_tc(zh, zt, rels3, rel_emb, sc_w1, sc_b1, w2row, sc_b2):
    return pl.pallas_call(
        _scorer_body,
        grid=(B // BQ,),
        in_specs=[
            pl.BlockSpec((BQ, D), lambda i: (i, 0)),
            pl.BlockSpec((BQ, D), lambda i: (i, 0)),
            pl.BlockSpec((1, 1, BQ), lambda i: (i, 0, 0)),
            pl.BlockSpec((R, D), lambda i: (0, 0)),
            pl.BlockSpec((3 * D, D), lambda i: (0, 0)),
            pl.BlockSpec((1, D), lambda i: (0, 0)),
            pl.BlockSpec((1, D), lambda i: (0, 0)),
            pl.BlockSpec((1, 1), lambda i: (0, 0)),
        ],
        out_specs=pl.BlockSpec((BQ, 1), lambda i: (i, 0)),
        out_shape=jax.ShapeDtypeStruct((B, 1), jnp.float32),
    )(zh, zt, rels3, rel_emb, sc_w1, sc_b1, w2row, sc_b2)


def kernel(node_feat, edge_index, edge_type, heads, rels, tails,
           W_in, b_in, rel_w0, sl_w0, sl_b0, rel_w1, sl_w1, sl_b1,
           rel_emb, sc_w1, sc_b1, sc_w2, sc_b2):
    src = edge_index[0]
    dst = edge_index[1]
    zpad = jnp.zeros((EPAD - E,), jnp.int32)
    src_p = jnp.concatenate([src, zpad])
    typ_p = jnp.concatenate([edge_type, zpad])
    dst_p = jnp.concatenate([dst, jnp.full((EPAD - E,), 2 * N, jnp.int32)])

    degp = _deg_sc(dst_p)

    x1 = _proj_tc(node_feat, W_in, b_in.reshape(1, D))
    ytab1, self1 = _ytab_tc(x1, rel_w0, sl_w0, sl_b0.reshape(1, D))
    agg1 = _agg_sc(ytab1.reshape(R * N, D), src_p, dst_p, typ_p)
    x2 = _comb_tc(self1, agg1, degp)

    ytab2, self2 = _ytab_tc(x2, rel_w1, sl_w1, sl_b1.reshape(1, D))
    agg2 = _agg_sc(ytab2.reshape(R * N, D), src_p, dst_p, typ_p)
    x3 = _comb_tc(self2, agg2, degp)

    zh, zt = _zgather_sc(x3, heads, tails)
    score = _scorer_tc(zh, zt, rels.reshape(B // BQ, 1, BQ), rel_emb,
                       sc_w1, sc_b1.reshape(1, D), sc_w2.reshape(1, D),
                       sc_b2.reshape(1, 1))
    return score.reshape(B)


# speedup vs baseline: 6.4176x; 6.4176x over previous
import functools

import jax
import jax.numpy as jnp
from jax import lax
from jax.experimental import pallas as pl
from jax.experimental.pallas import tpu as pltpu
from jax.experimental.pallas import tpu_sc as plsc

N = 50000
E = 800000
R = 8
D = 64
B = 4096

NC = 2
NS = 16

HALF = 25088
NPAD = 2 * HALF
HROWS = 25600
DUMMY = HALF
EPAD = 819200
EPC = EPAD // NS
CH = 2048
NCHUNK = EPC // CH
BN = 1000
BQ = 512

_HIGH = jax.lax.Precision.HIGHEST

_SC_PARAMS = pltpu.CompilerParams(use_tc_tiling_on_sc=False)


def _mesh():
    return plsc.VectorSubcoreMesh(core_axis_name="c", subcore_axis_name="s",
                                  num_cores=NC, num_subcores=NS)


def _deg_sc(dst_p):
    @functools.partial(
        pl.kernel,
        out_type=jax.ShapeDtypeStruct((NPAD, 16), jnp.float32),
        mesh=_mesh(),
        compiler_params=_SC_PARAMS,
        scratch_types=[
            pltpu.VMEM_SHARED((HROWS, 16), jnp.float32),
            pltpu.VMEM((CH,), jnp.int32),
            pltpu.VMEM((16, 128), jnp.int32),
            pltpu.VMEM((128, 16), jnp.float32),
            pltpu.VMEM((200, 16), jnp.float32),
        ],
    )
    def k(dst_hbm, deg_hbm, sdeg, dstb, locb, onesb, zb):
        cid = lax.axis_index("c")
        sid = lax.axis_index("s")
        base = cid * HALF
        ii = lax.iota(jnp.int32, 16)
        onerow = jnp.where(ii == 0, 1.0, 0.0).astype(jnp.float32)
        zrow = jnp.zeros((16,), jnp.float32)

        @pl.loop(0, 200)
        def _(i):
            zb[i, pl.ds(0, 16)] = zrow

        @pl.loop(0, 128)
        def _(i):
            onesb[i, pl.ds(0, 16)] = onerow

        @pl.loop(0, 8)
        def _(t):
            pltpu.sync_copy(zb, sdeg.at[pl.ds(sid * 1600 + t * 200, 200)])

        plsc.subcore_barrier()

        ebase = sid * EPC

        @pl.loop(0, NCHUNK)
        def _(t):
            pltpu.sync_copy(dst_hbm.at[pl.ds(ebase + t * CH, CH)], dstb)

            @pl.loop(0, 16)
            def _(j):
                @pl.loop(0, 8)
                def _(v):
                    dd = dstb[pl.ds(j * 128 + v * 16, 16)]
                    lo = dd - base
                    ok = (lo >= 0) & (lo < HALF)
                    locb[j, pl.ds(v * 16, 16)] = jnp.where(ok, lo, DUMMY)

            @pl.loop(0, 16)
            def _(j):
                pltpu.sync_copy(onesb, sdeg.at[locb.at[j]], add=True)

        plsc.subcore_barrier()
        pltpu.sync_copy(sdeg.at[pl.ds(sid * 1568, 1568)],
                        deg_hbm.at[pl.ds(base + sid * 1568, 1568)])

    return k(dst_p)


def _agg_sc(ytab, src_p, dst_p, typ_p):
    @functools.partial(
        pl.kernel,
        out_type=jax.ShapeDtypeStruct((NPAD, D), jnp.float32),
        mesh=_mesh(),
        compiler_params=_SC_PARAMS,
        scratch_types=[
            pltpu.VMEM_SHARED((HROWS, D), jnp.float32),
            pltpu.VMEM((CH,), jnp.int32),
            pltpu.VMEM((CH,), jnp.int32),
            pltpu.VMEM((CH,), jnp.int32),
            pltpu.VMEM((16, 128), jnp.int32),
            pltpu.VMEM((16, 128), jnp.int32),
            pltpu.VMEM((128, D), jnp.float32),
            pltpu.VMEM((100, D), jnp.float32),
            pltpu.SemaphoreType.DMA,
        ],
    )
    def k(ytab_hbm, src_hbm, dst_hbm, typ_hbm, agg_hbm,
          sagg, srcb, dstb, typb, gidx, locb, rowb, zb, sem):
        cid = lax.axis_index("c")
        sid = lax.axis_index("s")
        base = cid * HALF
        zrow = jnp.zeros((16,), jnp.float32)

        @pl.loop(0, 100)
        def _(i):
            @pl.loop(0, D // 16)
            def _(j):
                zb[i, pl.ds(j * 16, 16)] = zrow

        @pl.loop(0, 16)
        def _(t):
            pltpu.sync_copy(zb, sagg.at[pl.ds(sid * 1600 + t * 100, 100)])

        plsc.subcore_barrier()

        ebase = sid * EPC

        @pl.loop(0, NCHUNK)
        def _(t):
            off = ebase + t * CH
            pltpu.sync_copy(src_hbm.at[pl.ds(off, CH)], srcb)
            pltpu.sync_copy(typ_hbm.at[pl.ds(off, CH)], typb)
            pltpu.sync_copy(dst_hbm.at[pl.ds(off, CH)], dstb)

            @pl.loop(0, 16)
            def _(j):
                @pl.loop(0, 8)
                def _(v):
                    s = srcb[pl.ds(j * 128 + v * 16, 16)]
                    ty = typb[pl.ds(j * 128 + v * 16, 16)]
                    dd = dstb[pl.ds(j * 128 + v * 16, 16)]
                    gidx[j, pl.ds(v * 16, 16)] = ty * N + s
                    lo = dd - base
                    ok = (lo >= 0) & (lo < HALF)
                    locb[j, pl.ds(v * 16, 16)] = jnp.where(ok, lo, DUMMY)

            @pl.loop(0, 16)
            def _(j):
                pltpu.async_copy(ytab_hbm.at[gidx.at[j]], rowb, sem).wait()
                pltpu.sync_copy(rowb, sagg.at[locb.at[j]], add=True)

        plsc.subcore_barrier()
        pltpu.sync_copy(sagg.at[pl.ds(sid * 1568, 1568)],
                        agg_hbm.at[pl.ds(base + sid * 1568, 1568)])

    return k(ytab, src_p, dst_p, typ_p)


def _zgather_sc(x, heads, tails):
    rows = B // (NC * NS)

    @functools.partial(
        pl.kernel,
        out_type=(jax.ShapeDtypeStruct((B, D), jnp.float32),
                  jax.ShapeDtypeStruct((B, D), jnp.float32)),
        mesh=_mesh(),
        compiler_params=_SC_PARAMS,
        scratch_types=[
            pltpu.VMEM((rows,), jnp.int32),
            pltpu.VMEM((rows, D), jnp.float32),
            pltpu.SemaphoreType.DMA,
        ],
    )
    def k(x_hbm, h_hbm, t_hbm, zh_hbm, zt_hbm, idxb, rowb, sem):
        cid = lax.axis_index("c")
        sid = lax.axis_index("s")
        wid = sid * NC + cid
        off = wid * rows
        pltpu.sync_copy(h_hbm.at[pl.ds(off, rows)], idxb)
        pltpu.async_copy(x_hbm.at[idxb], rowb, sem).wait()
        pltpu.sync_copy(rowb, zh_hbm.at[pl.ds(off, rows)])
        pltpu.sync_copy(t_hbm.at[pl.ds(off, rows)], idxb)
        pltpu.async_copy(x_hbm.at[idxb], rowb, sem).wait()
        pltpu.sync_copy(rowb, zt_hbm.at[pl.ds(off, rows)])

    return k(x, heads, tails)


def _proj_body(nf_ref, w_ref, b_ref, o_ref):
    o_ref[...] = jnp.maximum(
        jnp.dot(nf_ref[...], w_ref[...], precision=_HIGH,
                preferred_element_type=jnp.float32) + b_ref[...], 0.0)


def _proj_tc(node_feat, W_in, b_in):
    return pl.pallas_call(
        _proj_body,
        grid=(N // BN,),
        in_specs=[
            pl.BlockSpec((BN, 4), lambda i: (i, 0)),
            pl.BlockSpec((4, D), lambda i: (0, 0)),
            pl.BlockSpec((1, D), lambda i: (0, 0)),
        ],
        out_specs=pl.BlockSpec((BN, D), lambda i: (i, 0)),
        out_shape=jax.ShapeDtypeStruct((N, D), jnp.float32),
    )(node_feat, W_in, b_in)


def _ytab_body(x_ref, rw_ref, sw_ref, sb_ref, y_ref, self_ref):
    xb = x_ref[...]
    y_ref[0] = jnp.dot(xb, rw_ref[0], precision=_HIGH,
                       preferred_element_type=jnp.float32)

    @pl.when(pl.program_id(1) == 0)
    def _():
        self_ref[...] = jnp.dot(xb, sw_ref[...], precision=_HIGH,
                                preferred_element_type=jnp.float32) + sb_ref[...]


def _ytab_tc(x, rel_w, sl_w, sl_b):
    return pl.pallas_call(
        _ytab_body,
        grid=(N // BN, R),
        in_specs=[
            pl.BlockSpec((BN, D), lambda i, r: (i, 0)),
            pl.BlockSpec((1, D, D), lambda i, r: (r, 0, 0)),
            pl.BlockSpec((D, D), lambda i, r: (0, 0)),
            pl.BlockSpec((1, D), lambda i, r: (0, 0)),
        ],
        out_specs=[
            pl.BlockSpec((1, BN, D), lambda i, r: (r, i, 0)),
            pl.BlockSpec((BN, D), lambda i, r: (i, 0)),
        ],
        out_shape=[
            jax.ShapeDtypeStruct((R, N, D), jnp.float32),
            jax.ShapeDtypeStruct((N, D), jnp.float32),
        ],
    )(x, rel_w, sl_w, sl_b)


def _comb_body(self_ref, agg_ref, deg_ref, o_ref):
    deg = deg_ref[:, 0:1]
    inv = 1.0 / jnp.maximum(deg, 1.0)
    o_ref[...] = jnp.maximum(self_ref[...] + agg_ref[...] * inv, 0.0)


def _comb_tc(self_x, agg, degp):
    return pl.pallas_call(
        _comb_body,
        grid=(N // BN,),
        in_specs=[
            pl.BlockSpec((BN, D), lambda i: (i, 0)),
            pl.BlockSpec((BN, D), lambda i: (i, 0)),
            pl.BlockSpec((BN, 16), lambda i: (i, 0)),
        ],
        out_specs=pl.BlockSpec((BN, D), lambda i: (i, 0)),
        out_shape=jax.ShapeDtypeStruct((N, D), jnp.float32),
    )(self_x, agg, degp)


def _scorer_body(zh_ref, zt_ref, rels_ref, re_ref, w1_ref, b1_ref, w2_ref,
                 b2_ref, o_ref):
    r = rels_ref[0, 0, :]
    oh = (lax.broadcasted_iota(jnp.int32, (BQ, R), 1) == r[:, None])
    re = jnp.dot(oh.astype(jnp.float32), re_ref[...], precision=_HIGH,
                 preferred_element_type=jnp.float32)
    w1 = w1_ref[...]
    h = (jnp.dot(zh_ref[...], w1[0:D], precision=_HIGH,
                 preferred_element_type=jnp.float32)
         + jnp.dot(zt_ref[...], w1[D:2 * D], precision=_HIGH,
                   preferred_element_type=jnp.float32)
         + jnp.dot(re, w1[2 * D:3 * D], precision=_HIGH,
                   preferred_element_type=jnp.float32)
         + b1_ref[...])
    h = jnp.maximum(h, 0.0)
    s = jnp.sum(h * w2_ref[...], axis=1, keepdims=True) + b2_ref[...]
    o_ref[...] = s


def _scorer_tc(zh, zt, rels3, rel_emb, sc_w1, sc_b1, w2row, sc_b2):
    return pl.pallas_call(
        _scorer_body,
        grid=(B // BQ,),
        in_specs=[
            pl.BlockSpec((BQ, D), lambda i: (i, 0)),
            pl.BlockSpec((BQ, D), lambda i: (i, 0)),
            pl.BlockSpec((1, 1, BQ), lambda i: (i, 0, 0)),
            pl.BlockSpec((R, D), lambda i: (0, 0)),
            pl.BlockSpec((3 * D, D), lambda i: (0, 0)),
            pl.BlockSpec((1, D), lambda i: (0, 0)),
            pl.BlockSpec((1, D), lambda i: (0, 0)),
            pl.BlockSpec((1, 1), lambda i: (0, 0)),
        ],
        out_specs=pl.BlockSpec((BQ, 1), lambda i: (i, 0)),
        out_shape=jax.ShapeDtypeStruct((B, 1), jnp.float32),
    )(zh, zt, rels3, rel_emb, sc_w1, sc_b1, w2row, sc_b2)


def kernel(node_feat, edge_index, edge_type, heads, rels, tails,
           W_in, b_in, rel_w0, sl_w0, sl_b0, rel_w1, sl_w1, sl_b1,
           rel_emb, sc_w1, sc_b1, sc_w2, sc_b2):
    src = edge_index[0]
    dst = edge_index[1]
    zpad = jnp.zeros((EPAD - E,), jnp.int32)
    src_p = jnp.concatenate([src, zpad])
    typ_p = jnp.concatenate([edge_type, zpad])
    dst_p = jnp.concatenate([dst, jnp.full((EPAD - E,), 2 * N, jnp.int32)])

    degp = _deg_sc(dst_p)

    x1 = _proj_tc(node_feat, W_in, b_in.reshape(1, D))
    ytab1, self1 = _ytab_tc(x1, rel_w0, sl_w0, sl_b0.reshape(1, D))
    agg1 = _agg_sc(ytab1.reshape(R * N, D), src_p, dst_p, typ_p)
    x2 = _comb_tc(self1, agg1, degp)

    ytab2, self2 = _ytab_tc(x2, rel_w1, sl_w1, sl_b1.reshape(1, D))
    agg2 = _agg_sc(ytab2.reshape(R * N, D), src_p, dst_p, typ_p)
    x3 = _comb_tc(self2, agg2, degp)

    zh, zt = _zgather_sc(x3, heads, tails)
    score = _scorer_tc(zh, zt, rels.reshape(B // BQ, 1, BQ), rel_emb,
                       sc_w1, sc_b1.reshape(1, D), sc_w2.reshape(1, D),
                       sc_b2.reshape(1, 1))
    return score.reshape(B)


# speedup vs baseline: 6.4495x; 1.0050x over previous
import functools

import jax
import jax.numpy as jnp
from jax import lax
from jax.experimental import pallas as pl
from jax.experimental.pallas import tpu as pltpu
from jax.experimental.pallas import tpu_sc as plsc

N = 50000
E = 800000
R = 8
D = 64
B = 4096

NC = 2
NS = 16

HALF = 25088
NPAD = 2 * HALF
HROWS = 25600
DUMMY = HALF
EPAD = 819200
EPC = EPAD // NS
CH = 2048
NCHUNK = EPC // CH
BN = 1000
BQ = 512

_HIGH = jax.lax.Precision.HIGHEST

_SC_PARAMS = pltpu.CompilerParams(use_tc_tiling_on_sc=False)


def _mesh():
    return plsc.VectorSubcoreMesh(core_axis_name="c", subcore_axis_name="s",
                                  num_cores=NC, num_subcores=NS)


def _deg_sc(dst_p):
    @functools.partial(
        pl.kernel,
        out_type=jax.ShapeDtypeStruct((NPAD, 16), jnp.float32),
        mesh=_mesh(),
        compiler_params=_SC_PARAMS,
        scratch_types=[
            pltpu.VMEM_SHARED((HROWS, 16), jnp.float32),
            pltpu.VMEM((CH,), jnp.int32),
            pltpu.VMEM((16, 128), jnp.int32),
            pltpu.VMEM((128, 16), jnp.float32),
            pltpu.VMEM((200, 16), jnp.float32),
        ],
    )
    def k(dst_hbm, deg_hbm, sdeg, dstb, locb, onesb, zb):
        cid = lax.axis_index("c")
        sid = lax.axis_index("s")
        base = cid * HALF
        ii = lax.iota(jnp.int32, 16)
        onerow = jnp.where(ii == 0, 1.0, 0.0).astype(jnp.float32)
        zrow = jnp.zeros((16,), jnp.float32)

        @pl.loop(0, 200)
        def _(i):
            zb[i, pl.ds(0, 16)] = zrow

        @pl.loop(0, 128)
        def _(i):
            onesb[i, pl.ds(0, 16)] = onerow

        @pl.loop(0, 8)
        def _(t):
            pltpu.sync_copy(zb, sdeg.at[pl.ds(sid * 1600 + t * 200, 200)])

        plsc.subcore_barrier()

        ebase = sid * EPC

        @pl.loop(0, NCHUNK)
        def _(t):
            pltpu.sync_copy(dst_hbm.at[pl.ds(ebase + t * CH, CH)], dstb)

            @pl.loop(0, 16)
            def _(j):
                @pl.loop(0, 8)
                def _(v):
                    dd = dstb[pl.ds(j * 128 + v * 16, 16)]
                    lo = dd - base
                    ok = (lo >= 0) & (lo < HALF)
                    locb[j, pl.ds(v * 16, 16)] = jnp.where(ok, lo, DUMMY)

            @pl.loop(0, 16)
            def _(j):
                pltpu.sync_copy(onesb, sdeg.at[locb.at[j]], add=True)

        plsc.subcore_barrier()
        pltpu.sync_copy(sdeg.at[pl.ds(sid * 1568, 1568)],
                        deg_hbm.at[pl.ds(base + sid * 1568, 1568)])

    return k(dst_p)


def _agg_sc(ytab, src_p, dst_p, typ_p):
    @functools.partial(
        pl.kernel,
        out_type=jax.ShapeDtypeStruct((NPAD, D), jnp.float32),
        mesh=_mesh(),
        compiler_params=_SC_PARAMS,
        scratch_types=[
            pltpu.VMEM_SHARED((HROWS, D), jnp.float32),
            pltpu.VMEM((CH,), jnp.int32),
            pltpu.VMEM((CH,), jnp.int32),
            pltpu.VMEM((CH,), jnp.int32),
            pltpu.VMEM((16, 128), jnp.int32),
            pltpu.VMEM((16, 128), jnp.int32),
            pltpu.VMEM((128, D), jnp.float32),
            pltpu.VMEM((128, D), jnp.float32),
            pltpu.SemaphoreType.DMA,
            pltpu.SemaphoreType.DMA,
        ],
    )
    def k(ytab_hbm, src_hbm, dst_hbm, typ_hbm, agg_hbm,
          sagg, srcb, dstb, typb, gidx, locb, rowb0, rowb1, sem0, sem1):
        cid = lax.axis_index("c")
        sid = lax.axis_index("s")
        base = cid * HALF
        zrow = jnp.zeros((16,), jnp.float32)

        @pl.loop(0, 128)
        def _(i):
            @pl.loop(0, D // 16)
            def _(j):
                rowb0[i, pl.ds(j * 16, 16)] = zrow

        @pl.loop(0, 12)
        def _(t):
            pltpu.sync_copy(rowb0, sagg.at[pl.ds(sid * 1600 + t * 128, 128)])

        pltpu.sync_copy(rowb0.at[pl.ds(0, 64)],
                        sagg.at[pl.ds(sid * 1600 + 1536, 64)])

        plsc.subcore_barrier()

        ebase = sid * EPC

        @pl.loop(0, NCHUNK)
        def _(t):
            off = ebase + t * CH
            pltpu.sync_copy(src_hbm.at[pl.ds(off, CH)], srcb)
            pltpu.sync_copy(typ_hbm.at[pl.ds(off, CH)], typb)
            pltpu.sync_copy(dst_hbm.at[pl.ds(off, CH)], dstb)

            @pl.loop(0, 16)
            def _(j):
                @pl.loop(0, 8)
                def _(v):
                    s = srcb[pl.ds(j * 128 + v * 16, 16)]
                    ty = typb[pl.ds(j * 128 + v * 16, 16)]
                    dd = dstb[pl.ds(j * 128 + v * 16, 16)]
                    gidx[j, pl.ds(v * 16, 16)] = ty * N + s
                    lo = dd - base
                    ok = (lo >= 0) & (lo < HALF)
                    locb[j, pl.ds(v * 16, 16)] = jnp.where(ok, lo, DUMMY)

            rbufs = (rowb0, rowb1)
            sems = (sem0, sem1)
            descs = [None, None]
            descs[0] = pltpu.async_copy(ytab_hbm.at[gidx.at[0]], rowb0, sem0)
            for j in range(16):
                bsel = j % 2
                descs[bsel].wait()
                if j + 1 < 16:
                    descs[1 - bsel] = pltpu.async_copy(
                        ytab_hbm.at[gidx.at[j + 1]], rbufs[1 - bsel],
                        sems[1 - bsel])
                pltpu.sync_copy(rbufs[bsel], sagg.at[locb.at[j]], add=True)

        plsc.subcore_barrier()
        pltpu.sync_copy(sagg.at[pl.ds(sid * 1568, 1568)],
                        agg_hbm.at[pl.ds(base + sid * 1568, 1568)])

    return k(ytab, src_p, dst_p, typ_p)


def _zgather_sc(x, heads, tails):
    rows = B // (NC * NS)

    @functools.partial(
        pl.kernel,
        out_type=(jax.ShapeDtypeStruct((B, D), jnp.float32),
                  jax.ShapeDtypeStruct((B, D), jnp.float32)),
        mesh=_mesh(),
        compiler_params=_SC_PARAMS,
        scratch_types=[
            pltpu.VMEM((rows,), jnp.int32),
            pltpu.VMEM((rows, D), jnp.float32),
            pltpu.SemaphoreType.DMA,
        ],
    )
    def k(x_hbm, h_hbm, t_hbm, zh_hbm, zt_hbm, idxb, rowb, sem):
        cid = lax.axis_index("c")
        sid = lax.axis_index("s")
        wid = sid * NC + cid
        off = wid * rows
        pltpu.sync_copy(h_hbm.at[pl.ds(off, rows)], idxb)
        pltpu.async_copy(x_hbm.at[idxb], rowb, sem).wait()
        pltpu.sync_copy(rowb, zh_hbm.at[pl.ds(off, rows)])
        pltpu.sync_copy(t_hbm.at[pl.ds(off, rows)], idxb)
        pltpu.async_copy(x_hbm.at[idxb], rowb, sem).wait()
        pltpu.sync_copy(rowb, zt_hbm.at[pl.ds(off, rows)])

    return k(x, heads, tails)


def _proj_body(nf_ref, w_ref, b_ref, o_ref):
    o_ref[...] = jnp.maximum(
        jnp.dot(nf_ref[...], w_ref[...], precision=_HIGH,
                preferred_element_type=jnp.float32) + b_ref[...], 0.0)


def _proj_tc(node_feat, W_in, b_in):
    return pl.pallas_call(
        _proj_body,
        grid=(N // BN,),
        in_specs=[
            pl.BlockSpec((BN, 4), lambda i: (i, 0)),
            pl.BlockSpec((4, D), lambda i: (0, 0)),
            pl.BlockSpec((1, D), lambda i: (0, 0)),
        ],
        out_specs=pl.BlockSpec((BN, D), lambda i: (i, 0)),
        out_shape=jax.ShapeDtypeStruct((N, D), jnp.float32),
    )(node_feat, W_in, b_in)


def _ytab_body(x_ref, rw_ref, sw_ref, sb_ref, y_ref, self_ref):
    xb = x_ref[...]
    y_ref[0] = jnp.dot(xb, rw_ref[0], precision=_HIGH,
                       preferred_element_type=jnp.float32)

    @pl.when(pl.program_id(1) == 0)
    def _():
        self_ref[...] = jnp.dot(xb, sw_ref[...], precision=_HIGH,
                                preferred_element_type=jnp.float32) + sb_ref[...]


def _ytab_tc(x, rel_w, sl_w, sl_b):
    return pl.pallas_call(
        _ytab_body,
        grid=(N // BN, R),
        in_specs=[
            pl.BlockSpec((BN, D), lambda i, r: (i, 0)),
            pl.BlockSpec((1, D, D), lambda i, r: (r, 0, 0)),
            pl.BlockSpec((D, D), lambda i, r: (0, 0)),
            pl.BlockSpec((1, D), lambda i, r: (0, 0)),
        ],
        out_specs=[
            pl.BlockSpec((1, BN, D), lambda i, r: (r, i, 0)),
            pl.BlockSpec((BN, D), lambda i, r: (i, 0)),
        ],
        out_shape=[
            jax.ShapeDtypeStruct((R, N, D), jnp.float32),
            jax.ShapeDtypeStruct((N, D), jnp.float32),
        ],
    )(x, rel_w, sl_w, sl_b)


def _comb_body(self_ref, agg_ref, deg_ref, o_ref):
    deg = deg_ref[:, 0:1]
    inv = 1.0 / jnp.maximum(deg, 1.0)
    o_ref[...] = jnp.maximum(self_ref[...] + agg_ref[...] * inv, 0.0)


def _comb_tc(self_x, agg, degp):
    return pl.pallas_call(
        _comb_body,
        grid=(N // BN,),
        in_specs=[
            pl.BlockSpec((BN, D), lambda i: (i, 0)),
            pl.BlockSpec((BN, D), lambda i: (i, 0)),
            pl.BlockSpec((BN, 16), lambda i: (i, 0)),
        ],
        out_specs=pl.BlockSpec((BN, D), lambda i: (i, 0)),
        out_shape=jax.ShapeDtypeStruct((N, D), jnp.float32),
    )(self_x, agg, degp)


def _scorer_body(zh_ref, zt_ref, rels_ref, re_ref, w1_ref, b1_ref, w2_ref,
                 b2_ref, o_ref):
    r = rels_ref[0, 0, :]
    oh = (lax.broadcasted_iota(jnp.int32, (BQ, R), 1) == r[:, None])
    re = jnp.dot(oh.astype(jnp.float32), re_ref[...], precision=_HIGH,
                 preferred_element_type=jnp.float32)
    w1 = w1_ref[...]
    h = (jnp.dot(zh_ref[...], w1[0:D], precision=_HIGH,
                 preferred_element_type=jnp.float32)
         + jnp.dot(zt_ref[...], w1[D:2 * D], precision=_HIGH,
                   preferred_element_type=jnp.float32)
         + jnp.dot(re, w1[2 * D:3 * D], precision=_HIGH,
                   preferred_element_type=jnp.float32)
         + b1_ref[...])
    h = jnp.maximum(h, 0.0)
    s = jnp.sum(h * w2_ref[...], axis=1, keepdims=True) + b2_ref[...]
    o_ref[...] = s


def _scorer_tc(zh, zt, rels3, rel_emb, sc_w1, sc_b1, w2row, sc_b2):
    return pl.pallas_call(
        _scorer_body,
        grid=(B // BQ,),
        in_specs=[
            pl.BlockSpec((BQ, D), lambda i: (i, 0)),
            pl.BlockSpec((BQ, D), lambda i: (i, 0)),
            pl.BlockSpec((1, 1, BQ), lambda i: (i, 0, 0)),
            pl.BlockSpec((R, D), lambda i: (0, 0)),
            pl.BlockSpec((3 * D, D), lambda i: (0, 0)),
            pl.BlockSpec((1, D), lambda i: (0, 0)),
            pl.BlockSpec((1, D), lambda i: (0, 0)),
            pl.BlockSpec((1, 1), lambda i: (0, 0)),
        ],
        out_specs=pl.BlockSpec((BQ, 1), lambda i: (i, 0)),
        out_shape=jax.ShapeDtypeStruct((B, 1), jnp.float32),
    )(zh, zt, rels3, rel_emb, sc_w1, sc_b1, w2row, sc_b2)


def kernel(node_feat, edge_index, edge_type, heads, rels, tails,
           W_in, b_in, rel_w0, sl_w0, sl_b0, rel_w1, sl_w1, sl_b1,
           rel_emb, sc_w1, sc_b1, sc_w2, sc_b2):
    src = edge_index[0]
    dst = edge_index[1]
    zpad = jnp.zeros((EPAD - E,), jnp.int32)
    src_p = jnp.concatenate([src, zpad])
    typ_p = jnp.concatenate([edge_type, zpad])
    dst_p = jnp.concatenate([dst, jnp.full((EPAD - E,), 2 * N, jnp.int32)])

    degp = _deg_sc(dst_p)

    x1 = _proj_tc(node_feat, W_in, b_in.reshape(1, D))
    ytab1, self1 = _ytab_tc(x1, rel_w0, sl_w0, sl_b0.reshape(1, D))
    agg1 = _agg_sc(ytab1.reshape(R * N, D), src_p, dst_p, typ_p)
    x2 = _comb_tc(self1, agg1, degp)

    ytab2, self2 = _ytab_tc(x2, rel_w1, sl_w1, sl_b1.reshape(1, D))
    agg2 = _agg_sc(ytab2.reshape(R * N, D), src_p, dst_p, typ_p)
    x3 = _comb_tc(self2, agg2, degp)

    zh, zt = _zgather_sc(x3, heads, tails)
    score = _scorer_tc(zh, zt, rels.reshape(B // BQ, 1, BQ), rel_emb,
                       sc_w1, sc_b1.reshape(1, D), sc_w2.reshape(1, D),
                       sc_b2.reshape(1, 1))
    return score.reshape(B)


# speedup vs baseline: 12.8655x; 1.9948x over previous
import functools

import jax
import jax.numpy as jnp
from jax import lax
from jax.experimental import pallas as pl
from jax.experimental.pallas import tpu as pltpu
from jax.experimental.pallas import tpu_sc as plsc

N = 50000
E = 800000
R = 8
D = 64
B = 4096

NC = 2
NS = 16

NP = 50176
RPS = NP // NS
DUMMY = 50100
HD = 32
EPAD = 819200
EPC = EPAD // NS
CH = 2048
NCHUNK = EPC // CH
CHD = 1024
EPCD = EPAD // (NC * NS)
NCHD = EPCD // CHD
BN = 1000
BQ = 512

_HIGH = jax.lax.Precision.HIGHEST

_SC_PARAMS = pltpu.CompilerParams(use_tc_tiling_on_sc=False)


def _mesh():
    return plsc.VectorSubcoreMesh(core_axis_name="c", subcore_axis_name="s",
                                  num_cores=NC, num_subcores=NS)


def _deg_sc(dst_p):
    @functools.partial(
        pl.kernel,
        out_type=jax.ShapeDtypeStruct((2 * NP, 16), jnp.float32),
        mesh=_mesh(),
        compiler_params=_SC_PARAMS,
        scratch_types=[
            pltpu.VMEM_SHARED((NP, 16), jnp.float32),
            pltpu.VMEM((CHD,), jnp.int32),
            pltpu.VMEM((8, 128), jnp.int32),
            pltpu.VMEM((128, 16), jnp.float32),
            pltpu.VMEM((196, 16), jnp.float32),
        ],
    )
    def k(dst_hbm, deg_hbm, sdeg, dstb, locb, onesb, zb):
        cid = lax.axis_index("c")
        sid = lax.axis_index("s")
        ii = lax.iota(jnp.int32, 16)
        onerow = jnp.where(ii == 0, 1.0, 0.0).astype(jnp.float32)
        zrow = jnp.zeros((16,), jnp.float32)

        @pl.loop(0, 196)
        def _(i):
            zb[i, pl.ds(0, 16)] = zrow

        @pl.loop(0, 128)
        def _(i):
            onesb[i, pl.ds(0, 16)] = onerow

        @pl.loop(0, 16)
        def _(t):
            pltpu.sync_copy(zb, sdeg.at[pl.ds(sid * RPS + t * 196, 196)])

        plsc.subcore_barrier()

        ebase = (cid * NS + sid) * EPCD

        @pl.loop(0, NCHD)
        def _(t):
            pltpu.sync_copy(dst_hbm.at[pl.ds(ebase + t * CHD, CHD)], dstb)

            @pl.loop(0, 8)
            def _(j):
                @pl.loop(0, 8)
                def _(v):
                    dd = dstb[pl.ds(j * 128 + v * 16, 16)]
                    locb[j, pl.ds(v * 16, 16)] = jnp.minimum(dd, DUMMY)

            @pl.loop(0, 8)
            def _(j):
                pltpu.sync_copy(onesb, sdeg.at[locb.at[j]], add=True)

        plsc.subcore_barrier()
        pltpu.sync_copy(sdeg.at[pl.ds(sid * RPS, RPS)],
                        deg_hbm.at[pl.ds(cid * NP + sid * RPS, RPS)])

    return k(dst_p)


def _agg_sc(ytab16, src_p, dst_p, typ_p):
    @functools.partial(
        pl.kernel,
        out_type=jax.ShapeDtypeStruct((2 * NP, HD), jnp.float32),
        mesh=_mesh(),
        compiler_params=_SC_PARAMS,
        scratch_types=[
            pltpu.VMEM_SHARED((NP, HD), jnp.float32),
            pltpu.VMEM((CH,), jnp.int32),
            pltpu.VMEM((CH,), jnp.int32),
            pltpu.VMEM((CH,), jnp.int32),
            pltpu.VMEM((16, 128), jnp.int32),
            pltpu.VMEM((16, 128), jnp.int32),
            pltpu.VMEM((128, HD), jnp.float32),
            pltpu.VMEM((128, HD), jnp.float32),
            pltpu.SemaphoreType.DMA,
            pltpu.SemaphoreType.DMA,
        ],
    )
    def k(ytab_hbm, src_hbm, dst_hbm, typ_hbm, agg_hbm,
          sagg, srcb, dstb, typb, gidx, locb, rowb0, rowb1, sem0, sem1):
        cid = lax.axis_index("c")
        sid = lax.axis_index("s")
        zrow = jnp.zeros((16,), jnp.float32)

        @pl.loop(0, 128)
        def _(i):
            @pl.loop(0, HD // 16)
            def _(j):
                rowb0[i, pl.ds(j * 16, 16)] = zrow

        @pl.loop(0, 24)
        def _(t):
            pltpu.sync_copy(rowb0, sagg.at[pl.ds(sid * RPS + t * 128, 128)])

        pltpu.sync_copy(rowb0.at[pl.ds(0, 64)],
                        sagg.at[pl.ds(sid * RPS + 3072, 64)])

        plsc.subcore_barrier()

        ebase = sid * EPC

        @pl.loop(0, NCHUNK)
        def _(t):
            off = ebase + t * CH
            pltpu.sync_copy(src_hbm.at[pl.ds(off, CH)], srcb)
            pltpu.sync_copy(typ_hbm.at[pl.ds(off, CH)], typb)
            pltpu.sync_copy(dst_hbm.at[pl.ds(off, CH)], dstb)

            @pl.loop(0, 16)
            def _(j):
                @pl.loop(0, 8)
                def _(v):
                    s = srcb[pl.ds(j * 128 + v * 16, 16)]
                    ty = typb[pl.ds(j * 128 + v * 16, 16)]
                    dd = dstb[pl.ds(j * 128 + v * 16, 16)]
                    gidx[j, pl.ds(v * 16, 16)] = s * 16 + ty * 2 + cid
                    locb[j, pl.ds(v * 16, 16)] = jnp.minimum(dd, DUMMY)

            rbufs = (rowb0, rowb1)
            sems = (sem0, sem1)
            descs = [None, None]
            descs[0] = pltpu.async_copy(ytab_hbm.at[gidx.at[0]], rowb0, sem0)
            for j in range(16):
                bsel = j % 2
                descs[bsel].wait()
                if j + 1 < 16:
                    descs[1 - bsel] = pltpu.async_copy(
                        ytab_hbm.at[gidx.at[j + 1]], rbufs[1 - bsel],
                        sems[1 - bsel])
                pltpu.sync_copy(rbufs[bsel], sagg.at[locb.at[j]], add=True)

        plsc.subcore_barrier()
        pltpu.sync_copy(sagg.at[pl.ds(sid * RPS, RPS)],
                        agg_hbm.at[pl.ds(cid * NP + sid * RPS, RPS)])

    return k(ytab16, src_p, dst_p, typ_p)


def _zgather_sc(x, heads, tails):
    rows = B // (NC * NS)

    @functools.partial(
        pl.kernel,
        out_type=(jax.ShapeDtypeStruct((B, D), jnp.float32),
                  jax.ShapeDtypeStruct((B, D), jnp.float32)),
        mesh=_mesh(),
        compiler_params=_SC_PARAMS,
        scratch_types=[
            pltpu.VMEM((rows,), jnp.int32),
            pltpu.VMEM((rows, D), jnp.float32),
            pltpu.SemaphoreType.DMA,
        ],
    )
    def k(x_hbm, h_hbm, t_hbm, zh_hbm, zt_hbm, idxb, rowb, sem):
        cid = lax.axis_index("c")
        sid = lax.axis_index("s")
        wid = sid * NC + cid
        off = wid * rows
        pltpu.sync_copy(h_hbm.at[pl.ds(off, rows)], idxb)
        pltpu.async_copy(x_hbm.at[idxb], rowb, sem).wait()
        pltpu.sync_copy(rowb, zh_hbm.at[pl.ds(off, rows)])
        pltpu.sync_copy(t_hbm.at[pl.ds(off, rows)], idxb)
        pltpu.async_copy(x_hbm.at[idxb], rowb, sem).wait()
        pltpu.sync_copy(rowb, zt_hbm.at[pl.ds(off, rows)])

    return k(x, heads, tails)


def _proj_body(nf_ref, w_ref, b_ref, o_ref):
    o_ref[...] = jnp.maximum(
        jnp.dot(nf_ref[...], w_ref[...], precision=_HIGH,
                preferred_element_type=jnp.float32) + b_ref[...], 0.0)


def _proj_tc(node_feat, W_in, b_in):
    return pl.pallas_call(
        _proj_body,
        grid=(N // BN,),
        in_specs=[
            pl.BlockSpec((BN, 4), lambda i: (i, 0)),
            pl.BlockSpec((4, D), lambda i: (0, 0)),
            pl.BlockSpec((1, D), lambda i: (0, 0)),
        ],
        out_specs=pl.BlockSpec((BN, D), lambda i: (i, 0)),
        out_shape=jax.ShapeDtypeStruct((N, D), jnp.float32),
    )(node_feat, W_in, b_in)


def _ytab_body(x_ref, wc_ref, sw_ref, sb_ref, y_ref, self_ref):
    xb = x_ref[...]
    y_ref[...] = jnp.dot(xb, wc_ref[...], precision=_HIGH,
                         preferred_element_type=jnp.float32)
    self_ref[...] = jnp.dot(xb, sw_ref[...], precision=_HIGH,
                            preferred_element_type=jnp.float32) + sb_ref[...]


def _ytab_tc(x, wcat, sl_w, sl_b):
    return pl.pallas_call(
        _ytab_body,
        grid=(N // BN,),
        in_specs=[
            pl.BlockSpec((BN, D), lambda i: (i, 0)),
            pl.BlockSpec((D, R * D), lambda i: (0, 0)),
            pl.BlockSpec((D, D), lambda i: (0, 0)),
            pl.BlockSpec((1, D), lambda i: (0, 0)),
        ],
        out_specs=[
            pl.BlockSpec((BN, R * D), lambda i: (i, 0)),
            pl.BlockSpec((BN, D), lambda i: (i, 0)),
        ],
        out_shape=[
            jax.ShapeDtypeStruct((N, R * D), jnp.float32),
            jax.ShapeDtypeStruct((N, D), jnp.float32),
        ],
    )(x, wcat, sl_w, sl_b)


def _comb_body(self_ref, a0_ref, a1_ref, d0_ref, d1_ref, o_ref):
    deg = d0_ref[:, 0:1] + d1_ref[:, 0:1]
    inv = 1.0 / jnp.maximum(deg, 1.0)
    agg = jnp.concatenate([a0_ref[...], a1_ref[...]], axis=1)
    o_ref[...] = jnp.maximum(self_ref[...] + agg * inv, 0.0)


def _comb_tc(self_x, a0, a1, d0, d1):
    return pl.pallas_call(
        _comb_body,
        grid=(N // BN,),
        in_specs=[
            pl.BlockSpec((BN, D), lambda i: (i, 0)),
            pl.BlockSpec((BN, HD), lambda i: (i, 0)),
            pl.BlockSpec((BN, HD), lambda i: (i, 0)),
            pl.BlockSpec((BN, 16), lambda i: (i, 0)),
            pl.BlockSpec((BN, 16), lambda i: (i, 0)),
        ],
        out_specs=pl.BlockSpec((BN, D), lambda i: (i, 0)),
        out_shape=jax.ShapeDtypeStruct((N, D), jnp.float32),
    )(self_x, a0, a1, d0, d1)


def _scorer_body(zh_ref, zt_ref, rels_ref, re_ref, w1_ref, b1_ref, w2_ref,
                 b2_ref, o_ref):
    r = rels_ref[0, 0, :]
    oh = (lax.broadcasted_iota(jnp.int32, (BQ, R), 1) == r[:, None])
    re = jnp.dot(oh.astype(jnp.float32), re_ref[...], precision=_HIGH,
                 preferred_element_type=jnp.float32)
    w1 = w1_ref[...]
    h = (jnp.dot(zh_ref[...], w1[0:D], precision=_HIGH,
                 preferred_element_type=jnp.float32)
         + jnp.dot(zt_ref[...], w1[D:2 * D], precision=_HIGH,
                   preferred_element_type=jnp.float32)
         + jnp.dot(re, w1[2 * D:3 * D], precision=_HIGH,
                   preferred_element_type=jnp.float32)
         + b1_ref[...])
    h = jnp.maximum(h, 0.0)
    s = jnp.sum(h * w2_ref[...], axis=1, keepdims=True) + b2_ref[...]
    o_ref[...] = s


def _scorer_tc(zh, zt, rels3, rel_emb, sc_w1, sc_b1, w2row, sc_b2):
    return pl.pallas_call(
        _scorer_body,
        grid=(B // BQ,),
        in_specs=[
            pl.BlockSpec((BQ, D), lambda i: (i, 0)),
            pl.BlockSpec((BQ, D), lambda i: (i, 0)),
            pl.BlockSpec((1, 1, BQ), lambda i: (i, 0, 0)),
            pl.BlockSpec((R, D), lambda i: (0, 0)),
            pl.BlockSpec((3 * D, D), lambda i: (0, 0)),
            pl.BlockSpec((1, D), lambda i: (0, 0)),
            pl.BlockSpec((1, D), lambda i: (0, 0)),
            pl.BlockSpec((1, 1), lambda i: (0, 0)),
        ],
        out_specs=pl.BlockSpec((BQ, 1), lambda i: (i, 0)),
        out_shape=jax.ShapeDtypeStruct((B, 1), jnp.float32),
    )(zh, zt, rels3, rel_emb, sc_w1, sc_b1, w2row, sc_b2)


def _layer(x, wcat, sl_w, sl_b, src_p, dst_p, typ_p, d0, d1):
    ycat, self_x = _ytab_tc(x, wcat, sl_w, sl_b)
    agg = _agg_sc(ycat.reshape(N * 16, HD), src_p, dst_p, typ_p)
    a0 = lax.slice(agg, (0, 0), (N, HD))
    a1 = lax.slice(agg, (NP, 0), (NP + N, HD))
    return _comb_tc(self_x, a0, a1, d0, d1)


def kernel(node_feat, edge_index, edge_type, heads, rels, tails,
           W_in, b_in, rel_w0, sl_w0, sl_b0, rel_w1, sl_w1, sl_b1,
           rel_emb, sc_w1, sc_b1, sc_w2, sc_b2):
    src = edge_index[0]
    dst = edge_index[1]
    zpad = jnp.zeros((EPAD - E,), jnp.int32)
    src_p = jnp.concatenate([src, zpad])
    typ_p = jnp.concatenate([edge_type, zpad])
    dst_p = jnp.concatenate([dst, jnp.full((EPAD - E,), 2 * N, jnp.int32)])

    wcat0 = rel_w0.transpose(1, 0, 2).reshape(D, R * D)
    wcat1 = rel_w1.transpose(1, 0, 2).reshape(D, R * D)

    degp = _deg_sc(dst_p)
    d0 = lax.slice(degp, (0, 0), (N, 16))
    d1 = lax.slice(degp, (NP, 0), (NP + N, 16))

    x1 = _proj_tc(node_feat, W_in, b_in.reshape(1, D))
    x2 = _layer(x1, wcat0, sl_w0, sl_b0.reshape(1, D), src_p, dst_p, typ_p,
                d0, d1)
    x3 = _layer(x2, wcat1, sl_w1, sl_b1.reshape(1, D), src_p, dst_p, typ_p,
                d0, d1)

    zh, zt = _zgather_sc(x3, heads, tails)
    score = _scorer_tc(zh, zt, rels.reshape(B // BQ, 1, BQ), rel_emb,
                       sc_w1, sc_b1.reshape(1, D), sc_w2.reshape(1, D),
                       sc_b2.reshape(1, 1))
    return score.reshape(B)


# speedup vs baseline: 14.1843x; 1.1025x over previous
import functools

import jax
import jax.numpy as jnp
from jax import lax
from jax.experimental import pallas as pl
from jax.experimental.pallas import tpu as pltpu
from jax.experimental.pallas import tpu_sc as plsc

N = 50000
E = 800000
R = 8
D = 64
B = 4096

NC = 2
NS = 16

NP = 51200
RPS = NP // NS
DUMMY = 50100
HD = 32
EPAD = 819200
EPC = EPAD // NS
CH = 2048
NCHUNK = EPC // CH
CHD = 1024
EPCD = EPAD // (NC * NS)
NCHD = EPCD // CHD
BN = 400
BQ = 512

_HIGH = jax.lax.Precision.HIGHEST

_SC_PARAMS = pltpu.CompilerParams(use_tc_tiling_on_sc=False)


def _mesh():
    return plsc.VectorSubcoreMesh(core_axis_name="c", subcore_axis_name="s",
                                  num_cores=NC, num_subcores=NS)


def _deg_sc(dst_p):
    @functools.partial(
        pl.kernel,
        out_type=jax.ShapeDtypeStruct((2 * NP, 16), jnp.float32),
        mesh=_mesh(),
        compiler_params=_SC_PARAMS,
        scratch_types=[
            pltpu.VMEM_SHARED((NP, 16), jnp.float32),
            pltpu.VMEM((CHD,), jnp.int32),
            pltpu.VMEM((8, 128), jnp.int32),
            pltpu.VMEM((128, 16), jnp.float32),
            pltpu.VMEM((128, 16), jnp.float32),
        ],
    )
    def k(dst_hbm, deg_hbm, sdeg, dstb, locb, onesb, zb):
        cid = lax.axis_index("c")
        sid = lax.axis_index("s")
        ii = lax.iota(jnp.int32, 16)
        onerow = jnp.where(ii == 0, 1.0, 0.0).astype(jnp.float32)
        zrow = jnp.zeros((16,), jnp.float32)

        @pl.loop(0, 128)
        def _(i):
            zb[i, pl.ds(0, 16)] = zrow
            onesb[i, pl.ds(0, 16)] = onerow

        @pl.loop(0, 25)
        def _(t):
            pltpu.sync_copy(zb, sdeg.at[pl.ds(sid * RPS + t * 128, 128)])

        plsc.subcore_barrier()

        ebase = (cid * NS + sid) * EPCD

        @pl.loop(0, NCHD)
        def _(t):
            pltpu.sync_copy(dst_hbm.at[pl.ds(ebase + t * CHD, CHD)], dstb)

            @pl.loop(0, 8)
            def _(j):
                @pl.loop(0, 8)
                def _(v):
                    dd = dstb[pl.ds(j * 128 + v * 16, 16)]
                    locb[j, pl.ds(v * 16, 16)] = jnp.minimum(dd, DUMMY)

            @pl.loop(0, 8)
            def _(j):
                pltpu.sync_copy(onesb, sdeg.at[locb.at[j]], add=True)

        plsc.subcore_barrier()
        pltpu.sync_copy(sdeg.at[pl.ds(sid * RPS, RPS)],
                        deg_hbm.at[pl.ds(cid * NP + sid * RPS, RPS)])

    return k(dst_p)


def _agg_sc(ytab16, src_p, dst_p, typ_p):
    @functools.partial(
        pl.kernel,
        out_type=jax.ShapeDtypeStruct((2 * NP, HD), jnp.float32),
        mesh=_mesh(),
        compiler_params=_SC_PARAMS,
        scratch_types=[
            pltpu.VMEM_SHARED((NP, HD), jnp.float32),
            pltpu.VMEM((CH,), jnp.int32),
            pltpu.VMEM((CH,), jnp.int32),
            pltpu.VMEM((CH,), jnp.int32),
            pltpu.VMEM((16, 128), jnp.int32),
            pltpu.VMEM((16, 128), jnp.int32),
            pltpu.VMEM((128, HD), jnp.float32),
            pltpu.VMEM((128, HD), jnp.float32),
            pltpu.SemaphoreType.DMA,
            pltpu.SemaphoreType.DMA,
            pltpu.SemaphoreType.DMA,
            pltpu.SemaphoreType.DMA,
        ],
    )
    def k(ytab_hbm, src_hbm, dst_hbm, typ_hbm, agg_hbm,
          sagg, srcb, dstb, typb, gidx, locb, rowb0, rowb1,
          gsem0, gsem1, ssem0, ssem1):
        cid = lax.axis_index("c")
        sid = lax.axis_index("s")
        zrow = jnp.zeros((16,), jnp.float32)

        @pl.loop(0, 128)
        def _(i):
            @pl.loop(0, HD // 16)
            def _(j):
                rowb0[i, pl.ds(j * 16, 16)] = zrow

        @pl.loop(0, 25)
        def _(t):
            pltpu.sync_copy(rowb0, sagg.at[pl.ds(sid * RPS + t * 128, 128)])

        plsc.subcore_barrier()

        ebase = sid * EPC

        @pl.loop(0, NCHUNK)
        def _(t):
            off = ebase + t * CH
            pltpu.sync_copy(src_hbm.at[pl.ds(off, CH)], srcb)
            pltpu.sync_copy(typ_hbm.at[pl.ds(off, CH)], typb)
            pltpu.sync_copy(dst_hbm.at[pl.ds(off, CH)], dstb)

            @pl.loop(0, 16)
            def _(j):
                @pl.loop(0, 8)
                def _(v):
                    s = srcb[pl.ds(j * 128 + v * 16, 16)]
                    ty = typb[pl.ds(j * 128 + v * 16, 16)]
                    dd = dstb[pl.ds(j * 128 + v * 16, 16)]
                    u = ty * 2 + cid
                    gidx[j, pl.ds(v * 16, 16)] = (
                        (u >> 2) * (4 * N) + s * 4 + (u & 3))
                    locb[j, pl.ds(v * 16, 16)] = jnp.minimum(dd, DUMMY)

            rbufs = (rowb0, rowb1)
            gsems = (gsem0, gsem1)
            ssems = (ssem0, ssem1)
            gdesc = [None, None]
            sdesc = [None, None]
            gdesc[0] = pltpu.async_copy(ytab_hbm.at[gidx.at[0]], rowb0, gsem0)
            for j in range(16):
                b = j % 2
                gdesc[b].wait()
                sdesc[b] = pltpu.async_copy(rbufs[b], sagg.at[locb.at[j]],
                                            ssems[b], add=True)
                if j + 1 < 16:
                    if sdesc[1 - b] is not None:
                        sdesc[1 - b].wait()
                    gdesc[1 - b] = pltpu.async_copy(
                        ytab_hbm.at[gidx.at[j + 1]], rbufs[1 - b],
                        gsems[1 - b])
            sdesc[0].wait()
            sdesc[1].wait()

        plsc.subcore_barrier()
        pltpu.sync_copy(sagg.at[pl.ds(sid * RPS, RPS)],
                        agg_hbm.at[pl.ds(cid * NP + sid * RPS, RPS)])

    return k(ytab16, src_p, dst_p, typ_p)


def _zgather_sc(x, heads, tails):
    rows = B // (NC * NS)

    @functools.partial(
        pl.kernel,
        out_type=(jax.ShapeDtypeStruct((B, 128), jnp.float32),
                  jax.ShapeDtypeStruct((B, 128), jnp.float32)),
        mesh=_mesh(),
        compiler_params=_SC_PARAMS,
        scratch_types=[
            pltpu.VMEM((rows,), jnp.int32),
            pltpu.VMEM((rows, 128), jnp.float32),
            pltpu.SemaphoreType.DMA,
        ],
    )
    def k(x_hbm, h_hbm, t_hbm, zh_hbm, zt_hbm, idxb, rowb, sem):
        cid = lax.axis_index("c")
        sid = lax.axis_index("s")
        wid = sid * NC + cid
        off = wid * rows
        pltpu.sync_copy(h_hbm.at[pl.ds(off, rows)], idxb)
        pltpu.async_copy(x_hbm.at[idxb], rowb, sem).wait()
        pltpu.sync_copy(rowb, zh_hbm.at[pl.ds(off, rows)])
        pltpu.sync_copy(t_hbm.at[pl.ds(off, rows)], idxb)
        pltpu.async_copy(x_hbm.at[idxb], rowb, sem).wait()
        pltpu.sync_copy(rowb, zt_hbm.at[pl.ds(off, rows)])

    return k(x, heads, tails)


def _l1_body(nf_ref, wi_ref, bi_ref, wc_ref, sw_ref, sb_ref, y_ref, self_ref):
    x = jnp.maximum(
        jnp.dot(nf_ref[...], wi_ref[...], precision=_HIGH,
                preferred_element_type=jnp.float32) + bi_ref[...], 0.0)
    for q in range(4):
        y_ref[q] = jnp.dot(x, wc_ref[:, 128 * q:128 * (q + 1)],
                           precision=_HIGH, preferred_element_type=jnp.float32)
    self_ref[...] = jnp.dot(x, sw_ref[...], precision=_HIGH,
                            preferred_element_type=jnp.float32) + sb_ref[...]


def _l1_tc(node_feat, W_in, b_in, wcat, sl_w, sl_b):
    return pl.pallas_call(
        _l1_body,
        grid=(N // BN,),
        in_specs=[
            pl.BlockSpec((BN, 4), lambda i: (i, 0)),
            pl.BlockSpec((4, D), lambda i: (0, 0)),
            pl.BlockSpec((1, D), lambda i: (0, 0)),
            pl.BlockSpec((D, R * D), lambda i: (0, 0)),
            pl.BlockSpec((D, D), lambda i: (0, 0)),
            pl.BlockSpec((1, D), lambda i: (0, 0)),
        ],
        out_specs=[
            pl.BlockSpec((4, BN, 128), lambda i: (0, i, 0)),
            pl.BlockSpec((BN, D), lambda i: (i, 0)),
        ],
        out_shape=[
            jax.ShapeDtypeStruct((4, N, 128), jnp.float32),
            jax.ShapeDtypeStruct((N, D), jnp.float32),
        ],
    )(node_feat, W_in, b_in.reshape(1, D), wcat, sl_w, sl_b.reshape(1, D))


def _comb(self_ref, a0_ref, a1_ref, d0_ref, d1_ref):
    deg = d0_ref[:, 0:1] + d1_ref[:, 0:1]
    inv = 1.0 / jnp.maximum(deg, 1.0)
    agg = jnp.concatenate([a0_ref[...], a1_ref[...]], axis=1)
    return jnp.maximum(self_ref[...] + agg * inv, 0.0)


def _l2_body(self_ref, a0_ref, a1_ref, d0_ref, d1_ref, wc_ref, sw_ref, sb_ref,
             y_ref, self2_ref):
    x = _comb(self_ref, a0_ref, a1_ref, d0_ref, d1_ref)
    for q in range(4):
        y_ref[q] = jnp.dot(x, wc_ref[:, 128 * q:128 * (q + 1)],
                           precision=_HIGH, preferred_element_type=jnp.float32)
    self2_ref[...] = jnp.dot(x, sw_ref[...], precision=_HIGH,
                             preferred_element_type=jnp.float32) + sb_ref[...]


def _agg_specs():
    return [
        pl.BlockSpec((BN, D), lambda i: (i, 0)),
        pl.BlockSpec((BN, HD), lambda i: (i, 0)),
        pl.BlockSpec((BN, HD), lambda i: (i + NP // BN, 0)),
        pl.BlockSpec((BN, 16), lambda i: (i, 0)),
        pl.BlockSpec((BN, 16), lambda i: (i + NP // BN, 0)),
    ]


def _l2_tc(self_x, agg, degp, wcat, sl_w, sl_b):
    return pl.pallas_call(
        _l2_body,
        grid=(N // BN,),
        in_specs=_agg_specs() + [
            pl.BlockSpec((D, R * D), lambda i: (0, 0)),
            pl.BlockSpec((D, D), lambda i: (0, 0)),
            pl.BlockSpec((1, D), lambda i: (0, 0)),
        ],
        out_specs=[
            pl.BlockSpec((4, BN, 128), lambda i: (0, i, 0)),
            pl.BlockSpec((BN, D), lambda i: (i, 0)),
        ],
        out_shape=[
            jax.ShapeDtypeStruct((4, N, 128), jnp.float32),
            jax.ShapeDtypeStruct((N, D), jnp.float32),
        ],
    )(self_x, agg, agg, degp, degp, wcat, sl_w, sl_b.reshape(1, D))


def _comb2_body(self_ref, a0_ref, a1_ref, d0_ref, d1_ref, o_ref):
    x = _comb(self_ref, a0_ref, a1_ref, d0_ref, d1_ref)
    o_ref[...] = jnp.concatenate(
        [x, jnp.zeros((BN, 128 - D), jnp.float32)], axis=1)


def _comb2_tc(self_x, agg, degp):
    return pl.pallas_call(
        _comb2_body,
        grid=(N // BN,),
        in_specs=_agg_specs(),
        out_specs=pl.BlockSpec((BN, 128), lambda i: (i, 0)),
        out_shape=jax.ShapeDtypeStruct((N, 128), jnp.float32),
    )(self_x, agg, agg, degp, degp)


def _scorer_body(zh_ref, zt_ref, rels_ref, re_ref, w1_ref, b1_ref, w2_ref,
                 b2_ref, o_ref):
    r = rels_ref[0, 0, :]
    oh = (lax.broadcasted_iota(jnp.int32, (BQ, R), 1) == r[:, None])
    re = jnp.dot(oh.astype(jnp.float32), re_ref[...], precision=_HIGH,
                 preferred_element_type=jnp.float32)
    w1 = w1_ref[...]
    h = (jnp.dot(zh_ref[:, 0:D], w1[0:D], precision=_HIGH,
                 preferred_element_type=jnp.float32)
         + jnp.dot(zt_ref[:, 0:D], w1[D:2 * D], precision=_HIGH,
                   preferred_element_type=jnp.float32)
         + jnp.dot(re, w1[2 * D:3 * D], precision=_HIGH,
                   preferred_element_type=jnp.float32)
         + b1_ref[...])
    h = jnp.maximum(h, 0.0)
    s = jnp.sum(h * w2_ref[...], axis=1, keepdims=True) + b2_ref[...]
    o_ref[...] = s


def _scorer_tc(zh, zt, rels3, rel_emb, sc_w1, sc_b1, w2row, sc_b2):
    return pl.pallas_call(
        _scorer_body,
        grid=(B // BQ,),
        in_specs=[
            pl.BlockSpec((BQ, 128), lambda i: (i, 0)),
            pl.BlockSpec((BQ, 128), lambda i: (i, 0)),
            pl.BlockSpec((1, 1, BQ), lambda i: (i, 0, 0)),
            pl.BlockSpec((R, D), lambda i: (0, 0)),
            pl.BlockSpec((3 * D, D), lambda i: (0, 0)),
            pl.BlockSpec((1, D), lambda i: (0, 0)),
            pl.BlockSpec((1, D), lambda i: (0, 0)),
            pl.BlockSpec((1, 1), lambda i: (0, 0)),
        ],
        out_specs=pl.BlockSpec((BQ, 1), lambda i: (i, 0)),
        out_shape=jax.ShapeDtypeStruct((B, 1), jnp.float32),
    )(zh, zt, rels3, rel_emb, sc_w1, sc_b1, w2row, sc_b2)


def kernel(node_feat, edge_index, edge_type, heads, rels, tails,
           W_in, b_in, rel_w0, sl_w0, sl_b0, rel_w1, sl_w1, sl_b1,
           rel_emb, sc_w1, sc_b1, sc_w2, sc_b2):
    src = edge_index[0]
    dst = edge_index[1]
    zpad = jnp.zeros((EPAD - E,), jnp.int32)
    src_p = jnp.concatenate([src, zpad])
    typ_p = jnp.concatenate([edge_type, zpad])
    dst_p = jnp.concatenate([dst, jnp.full((EPAD - E,), 2 * N, jnp.int32)])

    wcat0 = rel_w0.transpose(1, 0, 2).reshape(D, R * D)
    wcat1 = rel_w1.transpose(1, 0, 2).reshape(D, R * D)

    degp = _deg_sc(dst_p)

    y1, self1 = _l1_tc(node_feat, W_in, b_in, wcat0, sl_w0, sl_b0)
    agg1 = _agg_sc(y1.reshape(16 * N, HD), src_p, dst_p, typ_p)
    y2, self2 = _l2_tc(self1, agg1, degp, wcat1, sl_w1, sl_b1)
    agg2 = _agg_sc(y2.reshape(16 * N, HD), src_p, dst_p, typ_p)
    x3 = _comb2_tc(self2, agg2, degp)

    zh, zt = _zgather_sc(x3, heads, tails)
    score = _scorer_tc(zh, zt, rels.reshape(B // BQ, 1, BQ), rel_emb,
                       sc_w1, sc_b1.reshape(1, D), sc_w2.reshape(1, D),
                       sc_b2.reshape(1, 1))
    return score.reshape(B)


# speedup vs baseline: 15.0587x; 1.0616x over previous
import functools

import jax
import jax.numpy as jnp
from jax import lax
from jax.experimental import pallas as pl
from jax.experimental.pallas import tpu as pltpu
from jax.experimental.pallas import tpu_sc as plsc

N = 50000
E = 800000
R = 8
D = 64
B = 4096

NC = 2
NS = 16

NP = 51200
RPS = NP // NS
DUMMY = 50100
HD = 32
EPAD = 819200
EPC = EPAD // NS
CH = 2048
NCHUNK = EPC // CH
CHD = 1024
EPCD = EPAD // (NC * NS)
NCHD = EPCD // CHD
BN = 400
BQ = 512

_HIGH = None

_SC_PARAMS = pltpu.CompilerParams(use_tc_tiling_on_sc=False)


def _mesh():
    return plsc.VectorSubcoreMesh(core_axis_name="c", subcore_axis_name="s",
                                  num_cores=NC, num_subcores=NS)


def _deg_sc(dst_p):
    @functools.partial(
        pl.kernel,
        out_type=jax.ShapeDtypeStruct((2 * NP, 16), jnp.float32),
        mesh=_mesh(),
        compiler_params=_SC_PARAMS,
        scratch_types=[
            pltpu.VMEM_SHARED((NP, 16), jnp.float32),
            pltpu.VMEM((CHD,), jnp.int32),
            pltpu.VMEM((8, 128), jnp.int32),
            pltpu.VMEM((128, 16), jnp.float32),
            pltpu.VMEM((128, 16), jnp.float32),
        ],
    )
    def k(dst_hbm, deg_hbm, sdeg, dstb, locb, onesb, zb):
        cid = lax.axis_index("c")
        sid = lax.axis_index("s")
        ii = lax.iota(jnp.int32, 16)
        onerow = jnp.where(ii == 0, 1.0, 0.0).astype(jnp.float32)
        zrow = jnp.zeros((16,), jnp.float32)

        @pl.loop(0, 128)
        def _(i):
            zb[i, pl.ds(0, 16)] = zrow
            onesb[i, pl.ds(0, 16)] = onerow

        @pl.loop(0, 25)
        def _(t):
            pltpu.sync_copy(zb, sdeg.at[pl.ds(sid * RPS + t * 128, 128)])

        plsc.subcore_barrier()

        ebase = (cid * NS + sid) * EPCD

        @pl.loop(0, NCHD)
        def _(t):
            pltpu.sync_copy(dst_hbm.at[pl.ds(ebase + t * CHD, CHD)], dstb)

            @pl.loop(0, 8)
            def _(j):
                @pl.loop(0, 8)
                def _(v):
                    dd = dstb[pl.ds(j * 128 + v * 16, 16)]
                    locb[j, pl.ds(v * 16, 16)] = jnp.minimum(dd, DUMMY)

            @pl.loop(0, 8)
            def _(j):
                pltpu.sync_copy(onesb, sdeg.at[locb.at[j]], add=True)

        plsc.subcore_barrier()
        pltpu.sync_copy(sdeg.at[pl.ds(sid * RPS, RPS)],
                        deg_hbm.at[pl.ds(cid * NP + sid * RPS, RPS)])

    return k(dst_p)


def _agg_sc(ytab16, src_p, dst_p, typ_p):
    @functools.partial(
        pl.kernel,
        out_type=jax.ShapeDtypeStruct((2 * NP, HD), jnp.float32),
        mesh=_mesh(),
        compiler_params=_SC_PARAMS,
        scratch_types=[
            pltpu.VMEM_SHARED((NP, HD), jnp.float32),
            pltpu.VMEM((CH,), jnp.int32),
            pltpu.VMEM((CH,), jnp.int32),
            pltpu.VMEM((CH,), jnp.int32),
            pltpu.VMEM((16, 128), jnp.int32),
            pltpu.VMEM((16, 128), jnp.int32),
            pltpu.VMEM((128, HD), jnp.float32),
            pltpu.VMEM((128, HD), jnp.float32),
            pltpu.SemaphoreType.DMA,
            pltpu.SemaphoreType.DMA,
            pltpu.SemaphoreType.DMA,
            pltpu.SemaphoreType.DMA,
        ],
    )
    def k(ytab_hbm, src_hbm, dst_hbm, typ_hbm, agg_hbm,
          sagg, srcb, dstb, typb, gidx, locb, rowb0, rowb1,
          gsem0, gsem1, ssem0, ssem1):
        cid = lax.axis_index("c")
        sid = lax.axis_index("s")
        zrow = jnp.zeros((16,), jnp.float32)

        @pl.loop(0, 128)
        def _(i):
            @pl.loop(0, HD // 16)
            def _(j):
                rowb0[i, pl.ds(j * 16, 16)] = zrow

        @pl.loop(0, 25)
        def _(t):
            pltpu.sync_copy(rowb0, sagg.at[pl.ds(sid * RPS + t * 128, 128)])

        plsc.subcore_barrier()

        ebase = sid * EPC

        @pl.loop(0, NCHUNK)
        def _(t):
            off = ebase + t * CH
            pltpu.sync_copy(src_hbm.at[pl.ds(off, CH)], srcb)
            pltpu.sync_copy(typ_hbm.at[pl.ds(off, CH)], typb)
            pltpu.sync_copy(dst_hbm.at[pl.ds(off, CH)], dstb)

            @pl.loop(0, 16)
            def _(j):
                @pl.loop(0, 8)
                def _(v):
                    s = srcb[pl.ds(j * 128 + v * 16, 16)]
                    ty = typb[pl.ds(j * 128 + v * 16, 16)]
                    dd = dstb[pl.ds(j * 128 + v * 16, 16)]
                    u = ty * 2 + cid
                    gidx[j, pl.ds(v * 16, 16)] = (
                        (u >> 2) * (4 * N) + s * 4 + (u & 3))
                    locb[j, pl.ds(v * 16, 16)] = jnp.minimum(dd, DUMMY)

            rbufs = (rowb0, rowb1)
            gsems = (gsem0, gsem1)
            ssems = (ssem0, ssem1)
            gdesc = [None, None]
            sdesc = [None, None]
            gdesc[0] = pltpu.async_copy(ytab_hbm.at[gidx.at[0]], rowb0, gsem0)
            for j in range(16):
                b = j % 2
                gdesc[b].wait()
                sdesc[b] = pltpu.async_copy(rbufs[b], sagg.at[locb.at[j]],
                                            ssems[b], add=True)
                if j + 1 < 16:
                    if sdesc[1 - b] is not None:
                        sdesc[1 - b].wait()
                    gdesc[1 - b] = pltpu.async_copy(
                        ytab_hbm.at[gidx.at[j + 1]], rbufs[1 - b],
                        gsems[1 - b])
            sdesc[0].wait()
            sdesc[1].wait()

        plsc.subcore_barrier()
        pltpu.sync_copy(sagg.at[pl.ds(sid * RPS, RPS)],
                        agg_hbm.at[pl.ds(cid * NP + sid * RPS, RPS)])

    return k(ytab16, src_p, dst_p, typ_p)


def _zgather_sc(x, heads, tails):
    rows = B // (NC * NS)

    @functools.partial(
        pl.kernel,
        out_type=(jax.ShapeDtypeStruct((B, 128), jnp.float32),
                  jax.ShapeDtypeStruct((B, 128), jnp.float32)),
        mesh=_mesh(),
        compiler_params=_SC_PARAMS,
        scratch_types=[
            pltpu.VMEM((rows,), jnp.int32),
            pltpu.VMEM((rows, 128), jnp.float32),
            pltpu.SemaphoreType.DMA,
        ],
    )
    def k(x_hbm, h_hbm, t_hbm, zh_hbm, zt_hbm, idxb, rowb, sem):
        cid = lax.axis_index("c")
        sid = lax.axis_index("s")
        wid = sid * NC + cid
        off = wid * rows
        pltpu.sync_copy(h_hbm.at[pl.ds(off, rows)], idxb)
        pltpu.async_copy(x_hbm.at[idxb], rowb, sem).wait()
        pltpu.sync_copy(rowb, zh_hbm.at[pl.ds(off, rows)])
        pltpu.sync_copy(t_hbm.at[pl.ds(off, rows)], idxb)
        pltpu.async_copy(x_hbm.at[idxb], rowb, sem).wait()
        pltpu.sync_copy(rowb, zt_hbm.at[pl.ds(off, rows)])

    return k(x, heads, tails)


def _l1_body(nf_ref, wi_ref, bi_ref, wc_ref, sw_ref, sb_ref, y_ref, self_ref):
    x = jnp.maximum(
        jnp.dot(nf_ref[...], wi_ref[...], precision=_HIGH,
                preferred_element_type=jnp.float32) + bi_ref[...], 0.0)
    for q in range(4):
        y_ref[q] = jnp.dot(x, wc_ref[:, 128 * q:128 * (q + 1)],
                           precision=_HIGH, preferred_element_type=jnp.float32)
    self_ref[...] = jnp.dot(x, sw_ref[...], precision=_HIGH,
                            preferred_element_type=jnp.float32) + sb_ref[...]


def _l1_tc(node_feat, W_in, b_in, wcat, sl_w, sl_b):
    return pl.pallas_call(
        _l1_body,
        grid=(N // BN,),
        in_specs=[
            pl.BlockSpec((BN, 4), lambda i: (i, 0)),
            pl.BlockSpec((4, D), lambda i: (0, 0)),
            pl.BlockSpec((1, D), lambda i: (0, 0)),
            pl.BlockSpec((D, R * D), lambda i: (0, 0)),
            pl.BlockSpec((D, D), lambda i: (0, 0)),
            pl.BlockSpec((1, D), lambda i: (0, 0)),
        ],
        out_specs=[
            pl.BlockSpec((4, BN, 128), lambda i: (0, i, 0)),
            pl.BlockSpec((BN, D), lambda i: (i, 0)),
        ],
        out_shape=[
            jax.ShapeDtypeStruct((4, N, 128), jnp.float32),
            jax.ShapeDtypeStruct((N, D), jnp.float32),
        ],
    )(node_feat, W_in, b_in.reshape(1, D), wcat, sl_w, sl_b.reshape(1, D))


def _comb(self_ref, a0_ref, a1_ref, d0_ref, d1_ref):
    deg = d0_ref[:, 0:1] + d1_ref[:, 0:1]
    inv = 1.0 / jnp.maximum(deg, 1.0)
    agg = jnp.concatenate([a0_ref[...], a1_ref[...]], axis=1)
    return jnp.maximum(self_ref[...] + agg * inv, 0.0)


def _l2_body(self_ref, a0_ref, a1_ref, d0_ref, d1_ref, wc_ref, sw_ref, sb_ref,
             y_ref, self2_ref):
    x = _comb(self_ref, a0_ref, a1_ref, d0_ref, d1_ref)
    for q in range(4):
        y_ref[q] = jnp.dot(x, wc_ref[:, 128 * q:128 * (q + 1)],
                           precision=_HIGH, preferred_element_type=jnp.float32)
    self2_ref[...] = jnp.dot(x, sw_ref[...], precision=_HIGH,
                             preferred_element_type=jnp.float32) + sb_ref[...]


def _agg_specs():
    return [
        pl.BlockSpec((BN, D), lambda i: (i, 0)),
        pl.BlockSpec((BN, HD), lambda i: (i, 0)),
        pl.BlockSpec((BN, HD), lambda i: (i + NP // BN, 0)),
        pl.BlockSpec((BN, 16), lambda i: (i, 0)),
        pl.BlockSpec((BN, 16), lambda i: (i + NP // BN, 0)),
    ]


def _l2_tc(self_x, agg, degp, wcat, sl_w, sl_b):
    return pl.pallas_call(
        _l2_body,
        grid=(N // BN,),
        in_specs=_agg_specs() + [
            pl.BlockSpec((D, R * D), lambda i: (0, 0)),
            pl.BlockSpec((D, D), lambda i: (0, 0)),
            pl.BlockSpec((1, D), lambda i: (0, 0)),
        ],
        out_specs=[
            pl.BlockSpec((4, BN, 128), lambda i: (0, i, 0)),
            pl.BlockSpec((BN, D), lambda i: (i, 0)),
        ],
        out_shape=[
            jax.ShapeDtypeStruct((4, N, 128), jnp.float32),
            jax.ShapeDtypeStruct((N, D), jnp.float32),
        ],
    )(self_x, agg, agg, degp, degp, wcat, sl_w, sl_b.reshape(1, D))


def _comb2_body(self_ref, a0_ref, a1_ref, d0_ref, d1_ref, o_ref):
    x = _comb(self_ref, a0_ref, a1_ref, d0_ref, d1_ref)
    o_ref[...] = jnp.concatenate(
        [x, jnp.zeros((BN, 128 - D), jnp.float32)], axis=1)


def _comb2_tc(self_x, agg, degp):
    return pl.pallas_call(
        _comb2_body,
        grid=(N // BN,),
        in_specs=_agg_specs(),
        out_specs=pl.BlockSpec((BN, 128), lambda i: (i, 0)),
        out_shape=jax.ShapeDtypeStruct((N, 128), jnp.float32),
    )(self_x, agg, agg, degp, degp)


def _scorer_body(zh_ref, zt_ref, rels_ref, re_ref, w1_ref, b1_ref, w2_ref,
                 b2_ref, o_ref):
    r = rels_ref[0, 0, :]
    oh = (lax.broadcasted_iota(jnp.int32, (BQ, R), 1) == r[:, None])
    re = jnp.dot(oh.astype(jnp.float32), re_ref[...], precision=_HIGH,
                 preferred_element_type=jnp.float32)
    w1 = w1_ref[...]
    h = (jnp.dot(zh_ref[:, 0:D], w1[0:D], precision=_HIGH,
                 preferred_element_type=jnp.float32)
         + jnp.dot(zt_ref[:, 0:D], w1[D:2 * D], precision=_HIGH,
                   preferred_element_type=jnp.float32)
         + jnp.dot(re, w1[2 * D:3 * D], precision=_HIGH,
                   preferred_element_type=jnp.float32)
         + b1_ref[...])
    h = jnp.maximum(h, 0.0)
    s = jnp.sum(h * w2_ref[...], axis=1, keepdims=True) + b2_ref[...]
    o_ref[...] = s


def _scorer_tc(zh, zt, rels3, rel_emb, sc_w1, sc_b1, w2row, sc_b2):
    return pl.pallas_call(
        _scorer_body,
        grid=(B // BQ,),
        in_specs=[
            pl.BlockSpec((BQ, 128), lambda i: (i, 0)),
            pl.BlockSpec((BQ, 128), lambda i: (i, 0)),
            pl.BlockSpec((1, 1, BQ), lambda i: (i, 0, 0)),
            pl.BlockSpec((R, D), lambda i: (0, 0)),
            pl.BlockSpec((3 * D, D), lambda i: (0, 0)),
            pl.BlockSpec((1, D), lambda i: (0, 0)),
            pl.BlockSpec((1, D), lambda i: (0, 0)),
            pl.BlockSpec((1, 1), lambda i: (0, 0)),
        ],
        out_specs=pl.BlockSpec((BQ, 1), lambda i: (i, 0)),
        out_shape=jax.ShapeDtypeStruct((B, 1), jnp.float32),
    )(zh, zt, rels3, rel_emb, sc_w1, sc_b1, w2row, sc_b2)


def kernel(node_feat, edge_index, edge_type, heads, rels, tails,
           W_in, b_in, rel_w0, sl_w0, sl_b0, rel_w1, sl_w1, sl_b1,
           rel_emb, sc_w1, sc_b1, sc_w2, sc_b2):
    src = edge_index[0]
    dst = edge_index[1]
    zpad = jnp.zeros((EPAD - E,), jnp.int32)
    src_p = jnp.concatenate([src, zpad])
    typ_p = jnp.concatenate([edge_type, zpad])
    dst_p = jnp.concatenate([dst, jnp.full((EPAD - E,), 2 * N, jnp.int32)])

    wcat0 = rel_w0.transpose(1, 0, 2).reshape(D, R * D)
    wcat1 = rel_w1.transpose(1, 0, 2).reshape(D, R * D)

    degp = _deg_sc(dst_p)

    y1, self1 = _l1_tc(node_feat, W_in, b_in, wcat0, sl_w0, sl_b0)
    agg1 = _agg_sc(y1.reshape(16 * N, HD), src_p, dst_p, typ_p)
    y2, self2 = _l2_tc(self1, agg1, degp, wcat1, sl_w1, sl_b1)
    agg2 = _agg_sc(y2.reshape(16 * N, HD), src_p, dst_p, typ_p)
    x3 = _comb2_tc(self2, agg2, degp)

    zh, zt = _zgather_sc(x3, heads, tails)
    score = _scorer_tc(zh, zt, rels.reshape(B // BQ, 1, BQ), rel_emb,
                       sc_w1, sc_b1.reshape(1, D), sc_w2.reshape(1, D),
                       sc_b2.reshape(1, 1))
    return score.reshape(B)


# speedup vs baseline: 17.4905x; 1.1615x over previous
import functools

import jax
import jax.numpy as jnp
from jax import lax
from jax.experimental import pallas as pl
from jax.experimental.pallas import tpu as pltpu
from jax.experimental.pallas import tpu_sc as plsc

N = 50000
E = 800000
R = 8
D = 64
B = 4096

NC = 2
NS = 16

NP = 52000
RPS = NP // NS
DUMMY = 50100
HD = 32
EPAD = 819200
EPC = EPAD // NS
CH = 2048
NCHUNK = EPC // CH
CHD = 1024
EPCD = EPAD // (NC * NS)
NCHD = EPCD // CHD
BN = 2000
BQ = 512

_SC_PARAMS = pltpu.CompilerParams(use_tc_tiling_on_sc=False)


def _mesh():
    return plsc.VectorSubcoreMesh(core_axis_name="c", subcore_axis_name="s",
                                  num_cores=NC, num_subcores=NS)


def _deg_sc(dst_p):
    @functools.partial(
        pl.kernel,
        out_type=jax.ShapeDtypeStruct((2 * NP, 128), jnp.float32),
        mesh=_mesh(),
        compiler_params=_SC_PARAMS,
        scratch_types=[
            pltpu.VMEM_SHARED((NP, 16), jnp.float32),
            pltpu.VMEM((CHD,), jnp.int32),
            pltpu.VMEM((8, 128), jnp.int32),
            pltpu.VMEM((128, 16), jnp.float32),
            pltpu.VMEM((130, 16), jnp.float32),
        ],
    )
    def k(dst_hbm, deg_hbm, sdeg, dstb, locb, onesb, zb):
        cid = lax.axis_index("c")
        sid = lax.axis_index("s")
        ii = lax.iota(jnp.int32, 16)
        onerow = jnp.where(ii == 0, 1.0, 0.0).astype(jnp.float32)
        zrow = jnp.zeros((16,), jnp.float32)

        @pl.loop(0, 130)
        def _(i):
            zb[i, pl.ds(0, 16)] = zrow

        @pl.loop(0, 128)
        def _(i):
            onesb[i, pl.ds(0, 16)] = onerow

        @pl.loop(0, 25)
        def _(t):
            pltpu.sync_copy(zb, sdeg.at[pl.ds(sid * RPS + t * 130, 130)])

        plsc.subcore_barrier()

        ebase = (cid * NS + sid) * EPCD

        @pl.loop(0, NCHD)
        def _(t):
            pltpu.sync_copy(dst_hbm.at[pl.ds(ebase + t * CHD, CHD)], dstb)

            @pl.loop(0, 8)
            def _(j):
                @pl.loop(0, 8)
                def _(v):
                    dd = dstb[pl.ds(j * 128 + v * 16, 16)]
                    locb[j, pl.ds(v * 16, 16)] = jnp.minimum(dd, DUMMY)

            @pl.loop(0, 8)
            def _(j):
                pltpu.sync_copy(onesb, sdeg.at[locb.at[j]], add=True)

        plsc.subcore_barrier()
        pltpu.sync_copy(sdeg.at[pl.ds(sid * RPS, RPS)],
                        deg_hbm.at[pl.ds(cid * NP + sid * RPS, RPS),
                                   pl.ds(0, 16)])

    return k(dst_p)


def _agg_sc(ytab16, src_p, dst_p, typ_p):
    @functools.partial(
        pl.kernel,
        out_type=jax.ShapeDtypeStruct((2 * NP, 128), jnp.float32),
        mesh=_mesh(),
        compiler_params=_SC_PARAMS,
        scratch_types=[
            pltpu.VMEM_SHARED((NP, HD), jnp.float32),
            pltpu.VMEM((CH,), jnp.int32),
            pltpu.VMEM((CH,), jnp.int32),
            pltpu.VMEM((CH,), jnp.int32),
            pltpu.VMEM((16, 128), jnp.int32),
            pltpu.VMEM((16, 128), jnp.int32),
            pltpu.VMEM((128, HD), jnp.float32),
            pltpu.VMEM((128, HD), jnp.float32),
            pltpu.SemaphoreType.DMA,
            pltpu.SemaphoreType.DMA,
            pltpu.SemaphoreType.DMA,
            pltpu.SemaphoreType.DMA,
        ],
    )
    def k(ytab_hbm, src_hbm, dst_hbm, typ_hbm, agg_hbm,
          sagg, srcb, dstb, typb, gidx, locb, rowb0, rowb1,
          gsem0, gsem1, ssem0, ssem1):
        cid = lax.axis_index("c")
        sid = lax.axis_index("s")
        zrow = jnp.zeros((16,), jnp.float32)

        @pl.loop(0, 128)
        def _(i):
            @pl.loop(0, HD // 16)
            def _(j):
                rowb0[i, pl.ds(j * 16, 16)] = zrow

        @pl.loop(0, 26)
        def _(t):
            pltpu.sync_copy(rowb0.at[pl.ds(0, 125)],
                            sagg.at[pl.ds(sid * RPS + t * 125, 125)])

        plsc.subcore_barrier()

        ebase = sid * EPC

        @pl.loop(0, NCHUNK)
        def _(t):
            off = ebase + t * CH
            pltpu.sync_copy(src_hbm.at[pl.ds(off, CH)], srcb)
            pltpu.sync_copy(typ_hbm.at[pl.ds(off, CH)], typb)
            pltpu.sync_copy(dst_hbm.at[pl.ds(off, CH)], dstb)

            @pl.loop(0, 16)
            def _(j):
                @pl.loop(0, 8)
                def _(v):
                    s = srcb[pl.ds(j * 128 + v * 16, 16)]
                    ty = typb[pl.ds(j * 128 + v * 16, 16)]
                    dd = dstb[pl.ds(j * 128 + v * 16, 16)]
                    u = ty * 2 + cid
                    gidx[j, pl.ds(v * 16, 16)] = (
                        (u >> 2) * (4 * N) + s * 4 + (u & 3))
                    locb[j, pl.ds(v * 16, 16)] = jnp.minimum(dd, DUMMY)

            rbufs = (rowb0, rowb1)
            gsems = (gsem0, gsem1)
            ssems = (ssem0, ssem1)
            gdesc = [None, None]
            sdesc = [None, None]
            gdesc[0] = pltpu.async_copy(ytab_hbm.at[gidx.at[0]], rowb0, gsem0)
            for j in range(16):
                b = j % 2
                gdesc[b].wait()
                sdesc[b] = pltpu.async_copy(rbufs[b], sagg.at[locb.at[j]],
                                            ssems[b], add=True)
                if j + 1 < 16:
                    if sdesc[1 - b] is not None:
                        sdesc[1 - b].wait()
                    gdesc[1 - b] = pltpu.async_copy(
                        ytab_hbm.at[gidx.at[j + 1]], rbufs[1 - b],
                        gsems[1 - b])
            sdesc[0].wait()
            sdesc[1].wait()

        plsc.subcore_barrier()
        pltpu.sync_copy(sagg.at[pl.ds(sid * RPS, RPS)],
                        agg_hbm.at[pl.ds(cid * NP + sid * RPS, RPS),
                                   pl.ds(0, HD)])

    return k(ytab16, src_p, dst_p, typ_p)


def _zgather_sc(x, heads, tails):
    rows = B // (NC * NS)

    @functools.partial(
        pl.kernel,
        out_type=(jax.ShapeDtypeStruct((B, 128), jnp.float32),
                  jax.ShapeDtypeStruct((B, 128), jnp.float32)),
        mesh=_mesh(),
        compiler_params=_SC_PARAMS,
        scratch_types=[
            pltpu.VMEM((rows,), jnp.int32),
            pltpu.VMEM((rows, 128), jnp.float32),
            pltpu.SemaphoreType.DMA,
        ],
    )
    def k(x_hbm, h_hbm, t_hbm, zh_hbm, zt_hbm, idxb, rowb, sem):
        cid = lax.axis_index("c")
        sid = lax.axis_index("s")
        wid = sid * NC + cid
        off = wid * rows
        pltpu.sync_copy(h_hbm.at[pl.ds(off, rows)], idxb)
        pltpu.async_copy(x_hbm.at[idxb], rowb, sem).wait()
        pltpu.sync_copy(rowb, zh_hbm.at[pl.ds(off, rows)])
        pltpu.sync_copy(t_hbm.at[pl.ds(off, rows)], idxb)
        pltpu.async_copy(x_hbm.at[idxb], rowb, sem).wait()
        pltpu.sync_copy(rowb, zt_hbm.at[pl.ds(off, rows)])

    return k(x, heads, tails)


def _ytab_write(x, wc_ref, sw_ref, sb_ref, y_ref, self_ref):
    for q in range(4):
        y_ref[q] = jnp.dot(x, wc_ref[:, 128 * q:128 * (q + 1)],
                           preferred_element_type=jnp.float32)
    self_ref[...] = jnp.dot(x, sw_ref[...],
                            preferred_element_type=jnp.float32) + sb_ref[...]


def _l1_body(nf_ref, wi_ref, bi_ref, wc_ref, sw_ref, sb_ref, y_ref, self_ref):
    nf = nf_ref[...]
    acc = bi_ref[...]
    for kk in range(4):
        acc = acc + nf[:, kk:kk + 1] * wi_ref[kk:kk + 1, :]
    x = jnp.maximum(acc, 0.0)
    _ytab_write(x, wc_ref, sw_ref, sb_ref, y_ref, self_ref)


def _l1_tc(node_feat, W_in, b_in, wcat, sl_w, sl_b):
    return pl.pallas_call(
        _l1_body,
        grid=(N // BN,),
        in_specs=[
            pl.BlockSpec((BN, 4), lambda i: (i, 0)),
            pl.BlockSpec((4, D), lambda i: (0, 0)),
            pl.BlockSpec((1, D), lambda i: (0, 0)),
            pl.BlockSpec((D, R * D), lambda i: (0, 0)),
            pl.BlockSpec((D, D), lambda i: (0, 0)),
            pl.BlockSpec((1, D), lambda i: (0, 0)),
        ],
        out_specs=[
            pl.BlockSpec((4, BN, 128), lambda i: (0, i, 0)),
            pl.BlockSpec((BN, D), lambda i: (i, 0)),
        ],
        out_shape=[
            jax.ShapeDtypeStruct((4, N, 128), jnp.float32),
            jax.ShapeDtypeStruct((N, D), jnp.float32),
        ],
    )(node_feat, W_in, b_in.reshape(1, D), wcat, sl_w, sl_b.reshape(1, D))


def _comb(self_ref, a0_ref, a1_ref, d0_ref, d1_ref):
    deg = d0_ref[:, 0:1] + d1_ref[:, 0:1]
    inv = 1.0 / jnp.maximum(deg, 1.0)
    agg = jnp.concatenate([a0_ref[:, 0:HD], a1_ref[:, 0:HD]], axis=1)
    return jnp.maximum(self_ref[...] + agg * inv, 0.0)


def _l2_body(self_ref, a0_ref, a1_ref, d0_ref, d1_ref, wc_ref, sw_ref, sb_ref,
             y_ref, self2_ref):
    x = _comb(self_ref, a0_ref, a1_ref, d0_ref, d1_ref)
    _ytab_write(x, wc_ref, sw_ref, sb_ref, y_ref, self2_ref)


def _agg_specs():
    return [
        pl.BlockSpec((BN, D), lambda i: (i, 0)),
        pl.BlockSpec((BN, 128), lambda i: (i, 0)),
        pl.BlockSpec((BN, 128), lambda i: (i + NP // BN, 0)),
        pl.BlockSpec((BN, 128), lambda i: (i, 0)),
        pl.BlockSpec((BN, 128), lambda i: (i + NP // BN, 0)),
    ]


def _l2_tc(self_x, agg, degp, wcat, sl_w, sl_b):
    return pl.pallas_call(
        _l2_body,
        grid=(N // BN,),
        in_specs=_agg_specs() + [
            pl.BlockSpec((D, R * D), lambda i: (0, 0)),
            pl.BlockSpec((D, D), lambda i: (0, 0)),
            pl.BlockSpec((1, D), lambda i: (0, 0)),
        ],
        out_specs=[
            pl.BlockSpec((4, BN, 128), lambda i: (0, i, 0)),
            pl.BlockSpec((BN, D), lambda i: (i, 0)),
        ],
        out_shape=[
            jax.ShapeDtypeStruct((4, N, 128), jnp.float32),
            jax.ShapeDtypeStruct((N, D), jnp.float32),
        ],
    )(self_x, agg, agg, degp, degp, wcat, sl_w, sl_b.reshape(1, D))


def _comb2_body(self_ref, a0_ref, a1_ref, d0_ref, d1_ref, o_ref):
    x = _comb(self_ref, a0_ref, a1_ref, d0_ref, d1_ref)
    o_ref[...] = jnp.concatenate(
        [x, jnp.zeros((BN, 128 - D), jnp.float32)], axis=1)


def _comb2_tc(self_x, agg, degp):
    return pl.pallas_call(
        _comb2_body,
        grid=(N // BN,),
        in_specs=_agg_specs(),
        out_specs=pl.BlockSpec((BN, 128), lambda i: (i, 0)),
        out_shape=jax.ShapeDtypeStruct((N, 128), jnp.float32),
    )(self_x, agg, agg, degp, degp)


def _scorer_body(zh_ref, zt_ref, rels_ref, re_ref, w1_ref, b1_ref, w2_ref,
                 b2_ref, o_ref):
    r = rels_ref[0, 0, :]
    oh = (lax.broadcasted_iota(jnp.int32, (BQ, R), 1) == r[:, None])
    re = jnp.dot(oh.astype(jnp.float32), re_ref[...],
                 preferred_element_type=jnp.float32)
    w1 = w1_ref[...]
    h = (jnp.dot(zh_ref[:, 0:D], w1[0:D], preferred_element_type=jnp.float32)
         + jnp.dot(zt_ref[:, 0:D], w1[D:2 * D],
                   preferred_element_type=jnp.float32)
         + jnp.dot(re, w1[2 * D:3 * D], preferred_element_type=jnp.float32)
         + b1_ref[...])
    h = jnp.maximum(h, 0.0)
    s = jnp.sum(h * w2_ref[...], axis=1, keepdims=True) + b2_ref[...]
    o_ref[...] = s


def _scorer_tc(zh, zt, rels3, rel_emb, sc_w1, sc_b1, w2row, sc_b2):
    return pl.pallas_call(
        _scorer_body,
        grid=(B // BQ,),
        in_specs=[
            pl.BlockSpec((BQ, 128), lambda i: (i, 0)),
            pl.BlockSpec((BQ, 128), lambda i: (i, 0)),
            pl.BlockSpec((1, 1, BQ), lambda i: (i, 0, 0)),
            pl.BlockSpec((R, D), lambda i: (0, 0)),
            pl.BlockSpec((3 * D, D), lambda i: (0, 0)),
            pl.BlockSpec((1, D), lambda i: (0, 0)),
            pl.BlockSpec((1, D), lambda i: (0, 0)),
            pl.BlockSpec((1, 1), lambda i: (0, 0)),
        ],
        out_specs=pl.BlockSpec((BQ, 1), lambda i: (i, 0)),
        out_shape=jax.ShapeDtypeStruct((B, 1), jnp.float32),
    )(zh, zt, rels3, rel_emb, sc_w1, sc_b1, w2row, sc_b2)


def kernel(node_feat, edge_index, edge_type, heads, rels, tails,
           W_in, b_in, rel_w0, sl_w0, sl_b0, rel_w1, sl_w1, sl_b1,
           rel_emb, sc_w1, sc_b1, sc_w2, sc_b2):
    src = edge_index[0]
    dst = edge_index[1]
    zpad = jnp.zeros((EPAD - E,), jnp.int32)
    src_p = jnp.concatenate([src, zpad])
    typ_p = jnp.concatenate([edge_type, zpad])
    dst_p = jnp.concatenate([dst, jnp.full((EPAD - E,), 2 * N, jnp.int32)])

    wcat0 = rel_w0.transpose(1, 0, 2).reshape(D, R * D)
    wcat1 = rel_w1.transpose(1, 0, 2).reshape(D, R * D)

    degp = _deg_sc(dst_p)
    dst_a = dst_p + (0.0 * degp[0, 0]).astype(jnp.int32)

    y1, self1 = _l1_tc(node_feat, W_in, b_in, wcat0, sl_w0, sl_b0)
    agg1 = _agg_sc(y1.reshape(16 * N, HD), src_p, dst_a, typ_p)
    y2, self2 = _l2_tc(self1, agg1, degp, wcat1, sl_w1, sl_b1)
    agg2 = _agg_sc(y2.reshape(16 * N, HD), src_p, dst_p, typ_p)
    x3 = _comb2_tc(self2, agg2, degp)

    zh, zt = _zgather_sc(x3, heads, tails)
    score = _scorer_tc(zh, zt, rels.reshape(B // BQ, 1, BQ), rel_emb,
                       sc_w1, sc_b1.reshape(1, D), sc_w2.reshape(1, D),
                       sc_b2.reshape(1, 1))
    return score.reshape(B)


# speedup vs baseline: 19.3702x; 1.1075x over previous
import functools

import jax
import jax.numpy as jnp
from jax import lax
from jax.experimental import pallas as pl
from jax.experimental.pallas import tpu as pltpu
from jax.experimental.pallas import tpu_sc as plsc

N = 50000
E = 800000
R = 8
D = 64
B = 4096

NC = 2
NS = 16

NP = 52000
RPS = NP // NS
DUMMY = 50100
HD = 32
EPAD = 819200
EPC = EPAD // NS
CH = 2048
NCHUNK = EPC // CH
CHD = 1024
EPCD = EPAD // (NC * NS)
NCHD = EPCD // CHD
BN = 2000
BQ = 512

_SC_PARAMS = pltpu.CompilerParams(use_tc_tiling_on_sc=False)


def _mesh():
    return plsc.VectorSubcoreMesh(core_axis_name="c", subcore_axis_name="s",
                                  num_cores=NC, num_subcores=NS)


def _deg_sc(dst_p):
    @functools.partial(
        pl.kernel,
        out_type=jax.ShapeDtypeStruct((2 * NP, 128), jnp.float32),
        mesh=_mesh(),
        compiler_params=_SC_PARAMS,
        scratch_types=[
            pltpu.VMEM_SHARED((NP, 16), jnp.float32),
            pltpu.VMEM((CHD,), jnp.int32),
            pltpu.VMEM((8, 128), jnp.int32),
            pltpu.VMEM((128, 16), jnp.float32),
            pltpu.VMEM((130, 16), jnp.float32),
        ],
    )
    def k(dst_hbm, deg_hbm, sdeg, dstb, locb, onesb, zb):
        cid = lax.axis_index("c")
        sid = lax.axis_index("s")
        ii = lax.iota(jnp.int32, 16)
        onerow = jnp.where(ii == 0, 1.0, 0.0).astype(jnp.float32)
        zrow = jnp.zeros((16,), jnp.float32)

        @pl.loop(0, 130)
        def _(i):
            zb[i, pl.ds(0, 16)] = zrow

        @pl.loop(0, 128)
        def _(i):
            onesb[i, pl.ds(0, 16)] = onerow

        @pl.loop(0, 25)
        def _(t):
            pltpu.sync_copy(zb, sdeg.at[pl.ds(sid * RPS + t * 130, 130)])

        plsc.subcore_barrier()

        ebase = (cid * NS + sid) * EPCD

        @pl.loop(0, NCHD)
        def _(t):
            pltpu.sync_copy(dst_hbm.at[pl.ds(ebase + t * CHD, CHD)], dstb)

            @pl.loop(0, 8)
            def _(j):
                @pl.loop(0, 8)
                def _(v):
                    dd = dstb[pl.ds(j * 128 + v * 16, 16)]
                    locb[j, pl.ds(v * 16, 16)] = jnp.minimum(dd, DUMMY)

            @pl.loop(0, 8)
            def _(j):
                pltpu.sync_copy(onesb, sdeg.at[locb.at[j]], add=True)

        plsc.subcore_barrier()
        pltpu.sync_copy(sdeg.at[pl.ds(sid * RPS, RPS)],
                        deg_hbm.at[pl.ds(cid * NP + sid * RPS, RPS),
                                   pl.ds(0, 16)])

    return k(dst_p)


def _agg_sc(ytab16, eidx_p):
    NB = CH // 128

    @functools.partial(
        pl.kernel,
        out_type=jax.ShapeDtypeStruct((2 * NP, 128), jnp.float32),
        mesh=_mesh(),
        compiler_params=_SC_PARAMS,
        scratch_types=[
            pltpu.VMEM_SHARED((NP, HD), jnp.float32),
            pltpu.VMEM((3 * CH,), jnp.int32),
            pltpu.VMEM((16, 128), jnp.int32),
            pltpu.VMEM((16, 128), jnp.int32),
            pltpu.VMEM((128, HD), jnp.float32),
            pltpu.VMEM((128, HD), jnp.float32),
            pltpu.VMEM((128, HD), jnp.float32),
            pltpu.SemaphoreType.DMA,
            pltpu.SemaphoreType.DMA,
            pltpu.SemaphoreType.DMA,
            pltpu.SemaphoreType.DMA,
            pltpu.SemaphoreType.DMA,
            pltpu.SemaphoreType.DMA,
        ],
    )
    def k(ytab_hbm, eid_hbm, agg_hbm,
          sagg, eidb, gidx, locb, rowb0, rowb1, rowb2,
          gsem0, gsem1, gsem2, ssem0, ssem1, ssem2):
        cid = lax.axis_index("c")
        sid = lax.axis_index("s")
        zrow = jnp.zeros((16,), jnp.float32)

        @pl.loop(0, 128)
        def _(i):
            @pl.loop(0, HD // 16)
            def _(j):
                rowb0[i, pl.ds(j * 16, 16)] = zrow

        @pl.loop(0, 26)
        def _(t):
            pltpu.sync_copy(rowb0.at[pl.ds(0, 125)],
                            sagg.at[pl.ds(sid * RPS + t * 125, 125)])

        plsc.subcore_barrier()

        @pl.loop(0, NCHUNK)
        def _(t):
            pltpu.sync_copy(
                eid_hbm.at[pl.ds((sid * NCHUNK + t) * 3 * CH, 3 * CH)], eidb)

            @pl.loop(0, 16)
            def _(j):
                @pl.loop(0, 8)
                def _(v):
                    s = eidb[pl.ds(j * 128 + v * 16, 16)]
                    dd = eidb[pl.ds(CH + j * 128 + v * 16, 16)]
                    ty = eidb[pl.ds(2 * CH + j * 128 + v * 16, 16)]
                    u = ty * 2 + cid
                    gidx[j, pl.ds(v * 16, 16)] = (
                        (u >> 2) * (4 * N) + s * 4 + (u & 3))
                    locb[j, pl.ds(v * 16, 16)] = jnp.minimum(dd, DUMMY)

            rbufs = (rowb0, rowb1, rowb2)
            gsems = (gsem0, gsem1, gsem2)
            ssems = (ssem0, ssem1, ssem2)
            gdesc = [None, None, None]
            sdesc = [None, None, None]
            for p in range(2):
                gdesc[p] = pltpu.async_copy(ytab_hbm.at[gidx.at[p]],
                                            rbufs[p], gsems[p])
            for j in range(NB):
                b = j % 3
                gdesc[b].wait()
                sdesc[b] = pltpu.async_copy(rbufs[b], sagg.at[locb.at[j]],
                                            ssems[b], add=True)
                nj = j + 2
                if nj < NB:
                    nb = nj % 3
                    if sdesc[nb] is not None:
                        sdesc[nb].wait()
                        sdesc[nb] = None
                    gdesc[nb] = pltpu.async_copy(ytab_hbm.at[gidx.at[nj]],
                                                 rbufs[nb], gsems[nb])
            for b in range(3):
                if sdesc[b] is not None:
                    sdesc[b].wait()

        plsc.subcore_barrier()
        pltpu.sync_copy(sagg.at[pl.ds(sid * RPS, RPS)],
                        agg_hbm.at[pl.ds(cid * NP + sid * RPS, RPS),
                                   pl.ds(0, HD)])

    return k(ytab16, eidx_p)


def _zgather_sc(x, heads, tails):
    rows = B // (NC * NS)

    @functools.partial(
        pl.kernel,
        out_type=(jax.ShapeDtypeStruct((B, 128), jnp.float32),
                  jax.ShapeDtypeStruct((B, 128), jnp.float32)),
        mesh=_mesh(),
        compiler_params=_SC_PARAMS,
        scratch_types=[
            pltpu.VMEM((rows,), jnp.int32),
            pltpu.VMEM((rows, 128), jnp.float32),
            pltpu.SemaphoreType.DMA,
        ],
    )
    def k(x_hbm, h_hbm, t_hbm, zh_hbm, zt_hbm, idxb, rowb, sem):
        cid = lax.axis_index("c")
        sid = lax.axis_index("s")
        wid = sid * NC + cid
        off = wid * rows
        pltpu.sync_copy(h_hbm.at[pl.ds(off, rows)], idxb)
        pltpu.async_copy(x_hbm.at[idxb], rowb, sem).wait()
        pltpu.sync_copy(rowb, zh_hbm.at[pl.ds(off, rows)])
        pltpu.sync_copy(t_hbm.at[pl.ds(off, rows)], idxb)
        pltpu.async_copy(x_hbm.at[idxb], rowb, sem).wait()
        pltpu.sync_copy(rowb, zt_hbm.at[pl.ds(off, rows)])

    return k(x, heads, tails)


def _ytab_write(x, wc_ref, sw_ref, sb_ref, y_ref, self_ref):
    for q in range(4):
        y_ref[q] = jnp.dot(x, wc_ref[:, 128 * q:128 * (q + 1)],
                           preferred_element_type=jnp.float32)
    self_ref[...] = jnp.dot(x, sw_ref[...],
                            preferred_element_type=jnp.float32) + sb_ref[...]


def _l1_body(nf_ref, wi_ref, bi_ref, wc_ref, sw_ref, sb_ref, y_ref, self_ref):
    nf = nf_ref[...]
    acc = bi_ref[...]
    for kk in range(4):
        acc = acc + nf[:, kk:kk + 1] * wi_ref[kk:kk + 1, :]
    x = jnp.maximum(acc, 0.0)
    _ytab_write(x, wc_ref, sw_ref, sb_ref, y_ref, self_ref)


def _l1_tc(node_feat, W_in, b_in, wcat, sl_w, sl_b):
    return pl.pallas_call(
        _l1_body,
        grid=(N // BN,),
        in_specs=[
            pl.BlockSpec((BN, 4), lambda i: (i, 0)),
            pl.BlockSpec((4, D), lambda i: (0, 0)),
            pl.BlockSpec((1, D), lambda i: (0, 0)),
            pl.BlockSpec((D, R * D), lambda i: (0, 0)),
            pl.BlockSpec((D, D), lambda i: (0, 0)),
            pl.BlockSpec((1, D), lambda i: (0, 0)),
        ],
        out_specs=[
            pl.BlockSpec((4, BN, 128), lambda i: (0, i, 0)),
            pl.BlockSpec((BN, D), lambda i: (i, 0)),
        ],
        out_shape=[
            jax.ShapeDtypeStruct((4, N, 128), jnp.float32),
            jax.ShapeDtypeStruct((N, D), jnp.float32),
        ],
    )(node_feat, W_in, b_in.reshape(1, D), wcat, sl_w, sl_b.reshape(1, D))


def _comb(self_ref, a0_ref, a1_ref, d0_ref, d1_ref):
    deg = d0_ref[:, 0:1] + d1_ref[:, 0:1]
    inv = 1.0 / jnp.maximum(deg, 1.0)
    agg = jnp.concatenate([a0_ref[:, 0:HD], a1_ref[:, 0:HD]], axis=1)
    return jnp.maximum(self_ref[...] + agg * inv, 0.0)


def _l2_body(self_ref, a0_ref, a1_ref, d0_ref, d1_ref, wc_ref, sw_ref, sb_ref,
             y_ref, self2_ref):
    x = _comb(self_ref, a0_ref, a1_ref, d0_ref, d1_ref)
    _ytab_write(x, wc_ref, sw_ref, sb_ref, y_ref, self2_ref)


def _agg_specs():
    return [
        pl.BlockSpec((BN, D), lambda i: (i, 0)),
        pl.BlockSpec((BN, 128), lambda i: (i, 0)),
        pl.BlockSpec((BN, 128), lambda i: (i + NP // BN, 0)),
        pl.BlockSpec((BN, 128), lambda i: (i, 0)),
        pl.BlockSpec((BN, 128), lambda i: (i + NP // BN, 0)),
    ]


def _l2_tc(self_x, agg, degp, wcat, sl_w, sl_b):
    return pl.pallas_call(
        _l2_body,
        grid=(N // BN,),
        in_specs=_agg_specs() + [
            pl.BlockSpec((D, R * D), lambda i: (0, 0)),
            pl.BlockSpec((D, D), lambda i: (0, 0)),
            pl.BlockSpec((1, D), lambda i: (0, 0)),
        ],
        out_specs=[
            pl.BlockSpec((4, BN, 128), lambda i: (0, i, 0)),
            pl.BlockSpec((BN, D), lambda i: (i, 0)),
        ],
        out_shape=[
            jax.ShapeDtypeStruct((4, N, 128), jnp.float32),
            jax.ShapeDtypeStruct((N, D), jnp.float32),
        ],
    )(self_x, agg, agg, degp, degp, wcat, sl_w, sl_b.reshape(1, D))


def _comb2_body(self_ref, a0_ref, a1_ref, d0_ref, d1_ref, o_ref):
    x = _comb(self_ref, a0_ref, a1_ref, d0_ref, d1_ref)
    o_ref[...] = jnp.concatenate(
        [x, jnp.zeros((BN, 128 - D), jnp.float32)], axis=1)


def _comb2_tc(self_x, agg, degp):
    return pl.pallas_call(
        _comb2_body,
        grid=(N // BN,),
        in_specs=_agg_specs(),
        out_specs=pl.BlockSpec((BN, 128), lambda i: (i, 0)),
        out_shape=jax.ShapeDtypeStruct((N, 128), jnp.float32),
    )(self_x, agg, agg, degp, degp)


def _scorer_body(zh_ref, zt_ref, rels_ref, re_ref, w1_ref, b1_ref, w2_ref,
                 b2_ref, o_ref):
    r = rels_ref[0, 0, :]
    oh = (lax.broadcasted_iota(jnp.int32, (BQ, R), 1) == r[:, None])
    re = jnp.dot(oh.astype(jnp.float32), re_ref[...],
                 preferred_element_type=jnp.float32)
    w1 = w1_ref[...]
    h = (jnp.dot(zh_ref[:, 0:D], w1[0:D], preferred_element_type=jnp.float32)
         + jnp.dot(zt_ref[:, 0:D], w1[D:2 * D],
                   preferred_element_type=jnp.float32)
         + jnp.dot(re, w1[2 * D:3 * D], preferred_element_type=jnp.float32)
         + b1_ref[...])
    h = jnp.maximum(h, 0.0)
    s = jnp.sum(h * w2_ref[...], axis=1, keepdims=True) + b2_ref[...]
    o_ref[...] = s


def _scorer_tc(zh, zt, rels3, rel_emb, sc_w1, sc_b1, w2row, sc_b2):
    return pl.pallas_call(
        _scorer_body,
        grid=(B // BQ,),
        in_specs=[
            pl.BlockSpec((BQ, 128), lambda i: (i, 0)),
            pl.BlockSpec((BQ, 128), lambda i: (i, 0)),
            pl.BlockSpec((1, 1, BQ), lambda i: (i, 0, 0)),
            pl.BlockSpec((R, D), lambda i: (0, 0)),
            pl.BlockSpec((3 * D, D), lambda i: (0, 0)),
            pl.BlockSpec((1, D), lambda i: (0, 0)),
            pl.BlockSpec((1, D), lambda i: (0, 0)),
            pl.BlockSpec((1, 1), lambda i: (0, 0)),
        ],
        out_specs=pl.BlockSpec((BQ, 1), lambda i: (i, 0)),
        out_shape=jax.ShapeDtypeStruct((B, 1), jnp.float32),
    )(zh, zt, rels3, rel_emb, sc_w1, sc_b1, w2row, sc_b2)


def kernel(node_feat, edge_index, edge_type, heads, rels, tails,
           W_in, b_in, rel_w0, sl_w0, sl_b0, rel_w1, sl_w1, sl_b1,
           rel_emb, sc_w1, sc_b1, sc_w2, sc_b2):
    src = edge_index[0]
    dst = edge_index[1]
    zpad = jnp.zeros((EPAD - E,), jnp.int32)
    src_p = jnp.concatenate([src, zpad])
    typ_p = jnp.concatenate([edge_type, zpad])
    dst_p = jnp.concatenate([dst, jnp.full((EPAD - E,), 2 * N, jnp.int32)])
    eidx_p = (jnp.stack([src_p, dst_p, typ_p])
              .reshape(3, NS, NCHUNK, CH)
              .transpose(1, 2, 0, 3)
              .reshape(3 * EPAD))

    wcat0 = rel_w0.transpose(1, 0, 2).reshape(D, R * D)
    wcat1 = rel_w1.transpose(1, 0, 2).reshape(D, R * D)

    degp = _deg_sc(dst_p)
    eidx_a = eidx_p + (0.0 * degp[0, 0]).astype(jnp.int32)

    y1, self1 = _l1_tc(node_feat, W_in, b_in, wcat0, sl_w0, sl_b0)
    agg1 = _agg_sc(y1.reshape(16 * N, HD), eidx_a)
    y2, self2 = _l2_tc(self1, agg1, degp, wcat1, sl_w1, sl_b1)
    agg2 = _agg_sc(y2.reshape(16 * N, HD), eidx_p)
    x3 = _comb2_tc(self2, agg2, degp)

    zh, zt = _zgather_sc(x3, heads, tails)
    score = _scorer_tc(zh, zt, rels.reshape(B // BQ, 1, BQ), rel_emb,
                       sc_w1, sc_b1.reshape(1, D), sc_w2.reshape(1, D),
                       sc_b2.reshape(1, 1))
    return score.reshape(B)
